# baseline (speedup 1.0000x reference)
import functools

import jax
import jax.numpy as jnp
from jax import lax
from jax.experimental import pallas as pl
from jax.experimental.pallas import tpu as pltpu
from jax.experimental.pallas import tpu_sc as plsc

T = 2048
D = 1024
H = 16
KVH = 8
DH = 64
E = 8
TOPK = 2
FF = 512
SFF = 512
EPS = 1e-6
THETA = 1000000.0
LOG_THETA = 13.815510557964274

BT = 256
NT = T // BT
BG = 256
MAXB = 24
NPAD = MAXB * BG

INTERPRET = False

_f32 = jnp.float32
_bf16 = jnp.bfloat16


def _dot(a, b, dims):
    return jax.lax.dot_general(a, b, (dims, ((), ())),
                               preferred_element_type=_f32)


def _chunk_rmsnorm(x, w_full):
    w = x.shape[1]
    c = w // DH
    rr = jax.lax.broadcasted_iota(jnp.int32, (w, c), 0)
    cc = jax.lax.broadcasted_iota(jnp.int32, (w, c), 1)
    m = (rr // DH == cc).astype(_f32)
    ssq = _dot(x * x, m, ((1,), (0,)))
    scale = jax.lax.rsqrt(ssq / DH + EPS)
    scale_full = _dot(scale, m, ((1,), (1,)))
    return x * scale_full * w_full


def _rope(x, cs, sn):
    w = x.shape[1]
    lane = jax.lax.broadcasted_iota(jnp.int32, x.shape, 1)
    first = (lane % DH) < 32
    cosf = jnp.tile(cs, (1, w // 32))
    sinf = jnp.tile(sn, (1, w // 32))
    xp = jnp.concatenate([x[:, w - 32:], x[:, :w - 32]], axis=1)
    xm = jnp.concatenate([x[:, 32:], x[:, :32]], axis=1)
    return jnp.where(first, x * cosf - xm * sinf, xp * sinf + x * cosf)


def _pre_body(pos_ref, hid_ref, wqkv_ref, qnw_ref, knw_ref, ilnw_ref,
              q_ref, k_ref, v_ref):
    x = hid_ref[...]
    h = x * jax.lax.rsqrt(jnp.mean(x * x, axis=1, keepdims=True) + EPS)
    h = h * ilnw_ref[...]
    qkv = _dot(h.astype(_bf16), wqkv_ref[...], ((1,), (1,)))
    q = qkv[:, :H * DH]
    k = qkv[:, H * DH:(H + KVH) * DH]
    v = qkv[:, (H + KVH) * DH:]
    q = _chunk_rmsnorm(q, qnw_ref[...])
    k = _chunk_rmsnorm(k, knw_ref[...])
    posf = pos_ref[...].astype(_f32)
    jj = jax.lax.broadcasted_iota(jnp.int32, (posf.shape[0], 32), 1)
    f = posf * jnp.exp(jj.astype(_f32) * (-LOG_THETA / 32.0))
    cs = jnp.cos(f)
    sn = jnp.sin(f)
    q = _rope(q, cs, sn)
    k = _rope(k, cs, sn)
    q_ref[...] = q.astype(_bf16)
    k_ref[...] = k.astype(_bf16)
    v_ref[...] = v.astype(_bf16)


BTP = 512


def _pre(pos2d, hidden, wqkv_b, qnw, knw, ilnw):
    return pl.pallas_call(
        _pre_body,
        grid=(T // BTP,),
        in_specs=[
            pl.BlockSpec((BTP, 1), lambda i: (i, 0)),
            pl.BlockSpec((BTP, D), lambda i: (i, 0)),
            pl.BlockSpec(((H + 2 * KVH) * DH, D), lambda i: (0, 0)),
            pl.BlockSpec((1, H * DH), lambda i: (0, 0)),
            pl.BlockSpec((1, KVH * DH), lambda i: (0, 0)),
            pl.BlockSpec((1, D), lambda i: (0, 0)),
        ],
        out_specs=[
            pl.BlockSpec((BTP, H * DH), lambda i: (i, 0)),
            pl.BlockSpec((BTP, KVH * DH), lambda i: (i, 0)),
            pl.BlockSpec((BTP, KVH * DH), lambda i: (i, 0)),
        ],
        out_shape=[
            jax.ShapeDtypeStruct((T, H * DH), _bf16),
            jax.ShapeDtypeStruct((T, KVH * DH), _bf16),
            jax.ShapeDtypeStruct((T, KVH * DH), _bf16),
        ],
        compiler_params=pltpu.CompilerParams(
            dimension_semantics=("parallel",)),
        interpret=INTERPRET,
    )(pos2d, hidden, wqkv_b, qnw, knw, ilnw)


SCALE = DH ** -0.5
NHP = 4
NPAIRS = NT * (NT + 1) // 2


CS = 1024


def _attn_body(q_ref, k_ref, v_ref, o_ref):
    qi = pl.program_id(0)
    qs = [q_ref[:, DH * j:DH * (j + 1)] * 0.125 for j in range(H)]

    def chunk(base, carry, mask):
        out = []
        for j in range(H):
            kj = j // 2
            l_old, acc_old = carry[j]
            kh = k_ref[pl.ds(base, CS), DH * kj:DH * (kj + 1)]
            vh = v_ref[pl.ds(base, CS), DH * kj:DH * (kj + 1)]
            sc = _dot(qs[j], kh, ((1,), (1,)))
            if mask is not None:
                sc = sc + mask
            pm = jnp.exp(sc)
            l_new = l_old + jnp.sum(pm, axis=1, keepdims=True)
            acc_new = acc_old + _dot(pm.astype(_bf16), vh, ((1,), (0,)))
            out.append((l_new, acc_new))
        return tuple(out)

    init = tuple((jnp.zeros((BT, 1), _f32), jnp.zeros((BT, DH), _f32))
                 for _ in range(H))
    nfull = qi // (CS // BT)
    carry = jax.lax.fori_loop(
        0, nfull, lambda s, c: chunk(pl.multiple_of(s * CS, CS), c, None),
        init)
    rows = jax.lax.broadcasted_iota(jnp.int32, (BT, CS), 0) + qi * BT
    cols = jax.lax.broadcasted_iota(jnp.int32, (BT, CS), 1) + nfull * CS
    pen = jnp.where(rows >= cols, 0.0, -1e9).astype(_f32)
    carry = chunk(pl.multiple_of(nfull * CS, CS), carry, pen)
    for j in range(H):
        l_f, acc_f = carry[j]
        o_ref[:, DH * j:DH * (j + 1)] = (acc_f / l_f).astype(_bf16)


def _attn(q, k, v):
    return pl.pallas_call(
        _attn_body,
        grid=(NT,),
        in_specs=[
            pl.BlockSpec((BT, H * DH), lambda i: (i, 0)),
            pl.BlockSpec((T, KVH * DH), lambda i: (0, 0)),
            pl.BlockSpec((T, KVH * DH), lambda i: (0, 0)),
        ],
        out_specs=pl.BlockSpec((BT, H * DH), lambda i: (i, 0)),
        out_shape=jax.ShapeDtypeStruct((T, H * DH), _bf16),
        compiler_params=pltpu.CompilerParams(
            dimension_semantics=("arbitrary",)),
        interpret=INTERPRET,
    )(q, k, v)


def _mid_body(hid_ref, attn_ref, wo_ref, plnw_ref, wr_ref,
              x_ref, m_ref, lg_ref):
    x = hid_ref[...] + _dot(attn_ref[...], wo_ref[...], ((1,), (1,)))
    x_ref[...] = x
    m = x * jax.lax.rsqrt(jnp.mean(x * x, axis=1, keepdims=True) + EPS)
    m = m * plnw_ref[...]
    m_ref[...] = m
    lg_ref[...] = _dot(m, wr_ref[...], ((1,), (1,)))


def _mid(hidden, attn, wo_b, plnw, wr_pad):
    return pl.pallas_call(
        _mid_body,
        grid=(NT,),
        in_specs=[
            pl.BlockSpec((BT, D), lambda i: (i, 0)),
            pl.BlockSpec((BT, H * DH), lambda i: (i, 0)),
            pl.BlockSpec((D, H * DH), lambda i: (0, 0)),
            pl.BlockSpec((1, D), lambda i: (0, 0)),
            pl.BlockSpec((128, D), lambda i: (0, 0)),
        ],
        out_specs=[
            pl.BlockSpec((BT, D), lambda i: (i, 0)),
            pl.BlockSpec((BT, D), lambda i: (i, 0)),
            pl.BlockSpec((BT, 128), lambda i: (i, 0)),
        ],
        out_shape=[
            jax.ShapeDtypeStruct((T, D), _f32),
            jax.ShapeDtypeStruct((T, D), _f32),
            jax.ShapeDtypeStruct((T, 128), _f32),
        ],
        compiler_params=pltpu.CompilerParams(
            dimension_semantics=("parallel",)),
        interpret=INTERPRET,
    )(hidden, attn, wo_b, plnw, wr_pad)


def _cumsum_rows(x):
    n = x.shape[0]
    sh = 1
    while sh < n:
        z = jnp.zeros((sh, x.shape[1]), x.dtype)
        x = x + jnp.concatenate([z, x[:n - sh]], axis=0)
        sh *= 2
    return x


def _cumsum_lanes(x):
    n = x.shape[1]
    sh = 1
    while sh < n:
        z = jnp.zeros((x.shape[0], sh), x.dtype)
        x = x + jnp.concatenate([z, x[:, :n - sh]], axis=1)
        sh *= 2
    return x


def _route_body(lg_ref, w0_ref, w1_ref, e0_ref, e1_ref, d0_ref, d1_ref,
                be_ref, bv_ref):
    lane = jax.lax.broadcasted_iota(jnp.int32, (T, 128), 1)
    valid = lane < E
    l = jnp.where(valid, lg_ref[...], -1e30)
    m0 = jnp.max(l, axis=1, keepdims=True)
    e0 = jnp.min(jnp.where(l == m0, lane, 9999), axis=1, keepdims=True)
    oh0 = lane == e0
    l1 = jnp.where(oh0, -1e30, l)
    m1 = jnp.max(l1, axis=1, keepdims=True)
    e1 = jnp.min(jnp.where(l1 == m1, lane, 9999), axis=1, keepdims=True)
    oh1 = lane == e1
    w0_ref[...] = jax.nn.sigmoid(m0 - m1)
    w1_ref[...] = jax.nn.sigmoid(m1 - m0)
    e0_ref[...] = e0
    e1_ref[...] = e1
    f0 = oh0.astype(_f32)
    f1 = oh1.astype(_f32)
    c0 = _cumsum_rows(f0)
    c1 = _cumsum_rows(f1)
    cnt0 = jnp.sum(f0, axis=0, keepdims=True)
    cnt1 = jnp.sum(f1, axis=0, keepdims=True)
    ncnt = cnt0 + cnt1
    ru = jnp.ceil(ncnt / BG) * BG
    off = _cumsum_lanes(ru) - ru
    rank0 = jnp.sum((c0 - f0) * f0, axis=1, keepdims=True)
    rank1 = jnp.sum((cnt0 + c1 - f1) * f1, axis=1, keepdims=True)
    base0 = jnp.sum(off * f0, axis=1, keepdims=True)
    base1 = jnp.sum(off * f1, axis=1, keepdims=True)
    d0_ref[...] = (base0 + rank0).astype(jnp.int32)
    d1_ref[...] = (base1 + rank1).astype(jnp.int32)
    jb = jax.lax.broadcasted_iota(jnp.int32, (32, 128), 0).astype(_f32)
    lane32 = jax.lax.broadcasted_iota(jnp.int32, (32, 128), 1)
    bstart = off / BG
    bend = bstart + ru / BG
    ine = ((jb >= bstart) & (jb < bend) & (lane32 < E)).astype(_f32)
    beval = jnp.sum(ine * lane32.astype(_f32), axis=1, keepdims=True)
    has = jnp.sum(ine, axis=1, keepdims=True)
    lane1 = jax.lax.broadcasted_iota(jnp.int32, (1, 128), 1)
    last_e = jnp.max(jnp.where((ncnt > 0) & (lane1 < E),
                               lane1.astype(_f32), -1.0),
                     axis=1, keepdims=True)
    be_ref[...] = jnp.where(has > 0, beval, last_e).astype(jnp.int32)
    bv_ref[...] = (has > 0).astype(jnp.int32)


def _route(logits):
    return pl.pallas_call(
        _route_body,
        out_shape=[
            jax.ShapeDtypeStruct((T, 1), _f32),
            jax.ShapeDtypeStruct((T, 1), _f32),
            jax.ShapeDtypeStruct((T, 1), jnp.int32),
            jax.ShapeDtypeStruct((T, 1), jnp.int32),
            jax.ShapeDtypeStruct((T, 1), jnp.int32),
            jax.ShapeDtypeStruct((T, 1), jnp.int32),
            jax.ShapeDtypeStruct((32, 1), jnp.int32),
            jax.ShapeDtypeStruct((32, 1), jnp.int32),
        ],
        interpret=INTERPRET,
    )(logits)


def _moe_dense_body(m_ref, wgu_ref, wd_ref, e0_ref, e1_ref, w0_ref, w1_ref,
                    out_ref):
    e = pl.program_id(1)
    mb = m_ref[...].astype(_bf16)
    gu = _dot(mb, wgu_ref[...][0], ((1,), (0,)))
    hh = jax.nn.silu(gu[:, :FF]) * gu[:, FF:]
    y = _dot(hh.astype(_bf16), wd_ref[...][0], ((1,), (0,)))
    we = (jnp.where(e0_ref[...] == e, w0_ref[...], 0.0)
          + jnp.where(e1_ref[...] == e, w1_ref[...], 0.0))

    @pl.when(e == 0)
    def _():
        out_ref[...] = we * y

    @pl.when(e > 0)
    def _():
        out_ref[...] += we * y


def _moe_dense(m, wgu_b, wd_b, e0, e1, w0, w1):
    return pl.pallas_call(
        _moe_dense_body,
        grid=(NT, E),
        in_specs=[
            pl.BlockSpec((BT, D), lambda i, e: (i, 0)),
            pl.BlockSpec((1, D, 2 * FF), lambda i, e: (e, 0, 0)),
            pl.BlockSpec((1, FF, D), lambda i, e: (e, 0, 0)),
            pl.BlockSpec((BT, 1), lambda i, e: (i, 0)),
            pl.BlockSpec((BT, 1), lambda i, e: (i, 0)),
            pl.BlockSpec((BT, 1), lambda i, e: (i, 0)),
            pl.BlockSpec((BT, 1), lambda i, e: (i, 0)),
        ],
        out_specs=pl.BlockSpec((BT, D), lambda i, e: (i, 0)),
        out_shape=jax.ShapeDtypeStruct((T, D), _f32),
        compiler_params=pltpu.CompilerParams(
            dimension_semantics=("parallel", "arbitrary")),
        interpret=INTERPRET,
    )(m, wgu_b, wd_b, e0, e1, w0, w1)


RPW = T // 32


def _sc_scatter(m, d0, d1):
    mesh = plsc.VectorSubcoreMesh(core_axis_name="c", subcore_axis_name="s")

    @functools.partial(
        pl.kernel, mesh=mesh,
        out_type=jax.ShapeDtypeStruct((NPAD, D), _f32),
        scratch_types=[
            pltpu.VMEM((RPW,), jnp.int32),
            pltpu.VMEM((RPW,), jnp.int32),
            pltpu.VMEM((RPW, D), _f32),
            pltpu.SemaphoreType.DMA,
        ],
    )
    def k(m_hbm, d0_hbm, d1_hbm, xs_hbm, idx0_v, idx1_v, rows_v, sem):
        wid = lax.axis_index("c") * 16 + lax.axis_index("s")
        base = wid * RPW
        pltpu.sync_copy(d0_hbm.at[pl.ds(base, RPW)], idx0_v)
        pltpu.sync_copy(d1_hbm.at[pl.ds(base, RPW)], idx1_v)
        pltpu.sync_copy(m_hbm.at[pl.ds(base, RPW)], rows_v)
        pltpu.async_copy(rows_v, xs_hbm.at[idx0_v], sem).wait()
        pltpu.async_copy(rows_v, xs_hbm.at[idx1_v], sem).wait()

    return k(m, d0, d1)


def _sc_gather(ys, d0, d1):
    mesh = plsc.VectorSubcoreMesh(core_axis_name="c", subcore_axis_name="s")

    @functools.partial(
        pl.kernel, mesh=mesh,
        out_type=[
            jax.ShapeDtypeStruct((T, D), _f32),
            jax.ShapeDtypeStruct((T, D), _f32),
        ],
        scratch_types=[
            pltpu.VMEM((RPW,), jnp.int32),
            pltpu.VMEM((RPW, D), _f32),
            pltpu.SemaphoreType.DMA,
        ],
    )
    def k(ys_hbm, d0_hbm, d1_hbm, y0_hbm, y1_hbm, idx_v, rows_v, sem):
        wid = lax.axis_index("c") * 16 + lax.axis_index("s")
        base = wid * RPW
        pltpu.sync_copy(d0_hbm.at[pl.ds(base, RPW)], idx_v)
        pltpu.async_copy(ys_hbm.at[idx_v], rows_v, sem).wait()
        pltpu.sync_copy(rows_v, y0_hbm.at[pl.ds(base, RPW)])
        pltpu.sync_copy(d1_hbm.at[pl.ds(base, RPW)], idx_v)
        pltpu.async_copy(ys_hbm.at[idx_v], rows_v, sem).wait()
        pltpu.sync_copy(rows_v, y1_hbm.at[pl.ds(base, RPW)])

    return k(ys, d0, d1)


def _group_body(be_ref, bv_ref, xs_ref, wgu_ref, wd_ref, ys_ref):
    i = pl.program_id(0)

    @pl.when(bv_ref[i] > 0)
    def _():
        mb = xs_ref[...].astype(_bf16)
        gu = _dot(mb, wgu_ref[...][0], ((1,), (0,)))
        hh = jax.nn.silu(gu[:, :FF]) * gu[:, FF:]
        ys_ref[...] = _dot(hh.astype(_bf16), wd_ref[...][0], ((1,), (0,)))


def _group(xs, wgu_b, wd_b, be, bv):
    grid_spec = pltpu.PrefetchScalarGridSpec(
        num_scalar_prefetch=2,
        grid=(MAXB,),
        in_specs=[
            pl.BlockSpec((BG, D),
                         lambda i, be, bv: (jnp.where(bv[i] > 0, i, 0), 0)),
            pl.BlockSpec((1, D, 2 * FF), lambda i, be, bv: (be[i], 0, 0)),
            pl.BlockSpec((1, FF, D), lambda i, be, bv: (be[i], 0, 0)),
        ],
        out_specs=pl.BlockSpec((BG, D), lambda i, be, bv: (i, 0)),
    )
    return pl.pallas_call(
        _group_body,
        grid_spec=grid_spec,
        out_shape=jax.ShapeDtypeStruct((NPAD, D), _f32),
        compiler_params=pltpu.CompilerParams(
            dimension_semantics=("arbitrary",)),
        interpret=INTERPRET,
    )(be, bv, xs, wgu_b, wd_b)


def _shexp_body(x_ref, m_ref, wsgu_ref, wsd_ref, wsg_ref, out_ref):
    m = m_ref[...]
    sgu = _dot(m.astype(_bf16), wsgu_ref[...], ((1,), (0,)))
    sh = jax.nn.silu(sgu[:, :SFF]) * sgu[:, SFF:]
    sy = _dot(sh.astype(_bf16), wsd_ref[...], ((1,), (0,)))
    sg = jax.nn.sigmoid(jnp.sum(m * wsg_ref[...], axis=1, keepdims=True))
    out_ref[...] = x_ref[...] + sg * sy


def _shexp(x, m, wsgu_b, wsd_b, wsg):
    return pl.pallas_call(
        _shexp_body,
        grid=(NT,),
        in_specs=[
            pl.BlockSpec((BT, D), lambda i: (i, 0)),
            pl.BlockSpec((BT, D), lambda i: (i, 0)),
            pl.BlockSpec((D, 2 * SFF), lambda i: (0, 0)),
            pl.BlockSpec((SFF, D), lambda i: (0, 0)),
            pl.BlockSpec((1, D), lambda i: (0, 0)),
        ],
        out_specs=pl.BlockSpec((BT, D), lambda i: (i, 0)),
        out_shape=jax.ShapeDtypeStruct((T, D), _f32),
        compiler_params=pltpu.CompilerParams(
            dimension_semantics=("parallel",)),
        interpret=INTERPRET,
    )(x, m, wsgu_b, wsd_b, wsg)


def _comb_body(xs_ref, y0_ref, y1_ref, w0_ref, w1_ref, out_ref):
    fused = w0_ref[...] * y0_ref[...] + w1_ref[...] * y1_ref[...]
    out_ref[...] = xs_ref[...] + fused


def _comb(xsh, y0, y1, w0, w1):
    return pl.pallas_call(
        _comb_body,
        grid=(NT,),
        in_specs=[
            pl.BlockSpec((BT, D), lambda i: (i, 0)),
            pl.BlockSpec((BT, D), lambda i: (i, 0)),
            pl.BlockSpec((BT, D), lambda i: (i, 0)),
            pl.BlockSpec((BT, 1), lambda i: (i, 0)),
            pl.BlockSpec((BT, 1), lambda i: (i, 0)),
        ],
        out_specs=pl.BlockSpec((BT, D), lambda i: (i, 0)),
        out_shape=jax.ShapeDtypeStruct((T, D), _f32),
        compiler_params=pltpu.CompilerParams(
            dimension_semantics=("parallel",)),
        interpret=INTERPRET,
    )(xsh, y0, y1, w0, w1)


def kernel(positions, hidden_states, w_qkv, w_o, q_norm_w, k_norm_w,
           input_ln_w, post_ln_w, w_router, w_gate_up, w_down,
           w_shared_gu, w_shared_down, w_shared_gate):
    pos2d = positions.reshape(T, 1).astype(jnp.int32)
    qnw = jnp.tile(q_norm_w, H).reshape(1, H * DH)
    knw = jnp.tile(k_norm_w, KVH).reshape(1, KVH * DH)
    ilnw = input_ln_w.reshape(1, D)
    plnw = post_ln_w.reshape(1, D)
    wr_pad = jnp.pad(w_router, ((0, 128 - E), (0, 0)))
    wqkv_b = w_qkv.astype(_bf16)
    wo_b = w_o.astype(_bf16)
    wgu_b = w_gate_up.astype(_bf16)
    wd_b = w_down.astype(_bf16)
    wsgu_b = w_shared_gu.astype(_bf16)
    wsd_b = w_shared_down.astype(_bf16)
    wsg = w_shared_gate.reshape(1, D)

    q, k, v = _pre(pos2d, hidden_states, wqkv_b, qnw, knw, ilnw)
    attn = _attn(q, k, v)

    x, m, logits = _mid(hidden_states, attn, wo_b, plnw, wr_pad)
    w0, w1, e0, e1, d0, d1, be, bv = _route(logits)
    d0f = d0.reshape(T)
    d1f = d1.reshape(T)
    bef = be.reshape(32)[:MAXB]
    bvf = bv.reshape(32)[:MAXB]
    xs = _sc_scatter(m, d0f, d1f)
    xsh = _shexp(x, m, wsgu_b, wsd_b, wsg)
    ys = _group(xs, wgu_b, wd_b, bef, bvf)
    y0, y1 = _sc_gather(ys, d0f, d1f)
    out = _comb(xsh, y0, y1, w0, w1)
    return out


# speedup vs baseline: 1.0186x; 1.0186x over previous
import functools

import jax
import jax.numpy as jnp
from jax import lax
from jax.experimental import pallas as pl
from jax.experimental.pallas import tpu as pltpu
from jax.experimental.pallas import tpu_sc as plsc

T = 2048
D = 1024
H = 16
KVH = 8
DH = 64
E = 8
TOPK = 2
FF = 512
SFF = 512
EPS = 1e-6
THETA = 1000000.0
LOG_THETA = 13.815510557964274

BT = 256
NT = T // BT
BG = 256
MAXB = 24
NPAD = MAXB * BG

INTERPRET = False

_f32 = jnp.float32
_bf16 = jnp.bfloat16


def _dot(a, b, dims):
    return jax.lax.dot_general(a, b, (dims, ((), ())),
                               preferred_element_type=_f32)


def _chunk_rmsnorm(x, w_full):
    w = x.shape[1]
    c = w // DH
    rr = jax.lax.broadcasted_iota(jnp.int32, (w, c), 0)
    cc = jax.lax.broadcasted_iota(jnp.int32, (w, c), 1)
    m = (rr // DH == cc).astype(_f32)
    ssq = _dot(x * x, m, ((1,), (0,)))
    scale = jax.lax.rsqrt(ssq / DH + EPS)
    scale_full = _dot(scale, m, ((1,), (1,)))
    return x * scale_full * w_full


def _rope(x, cs, sn):
    w = x.shape[1]
    lane = jax.lax.broadcasted_iota(jnp.int32, x.shape, 1)
    first = (lane % DH) < 32
    cosf = jnp.tile(cs, (1, w // 32))
    sinf = jnp.tile(sn, (1, w // 32))
    xp = jnp.concatenate([x[:, w - 32:], x[:, :w - 32]], axis=1)
    xm = jnp.concatenate([x[:, 32:], x[:, :32]], axis=1)
    return jnp.where(first, x * cosf - xm * sinf, xp * sinf + x * cosf)


def _pre_body(pos_ref, hid_ref, wqkv_ref, qnw_ref, knw_ref, ilnw_ref,
              q_ref, k_ref, v_ref):
    x = hid_ref[...]
    h = x * jax.lax.rsqrt(jnp.mean(x * x, axis=1, keepdims=True) + EPS)
    h = h * ilnw_ref[...]
    qkv = _dot(h.astype(_bf16), wqkv_ref[...], ((1,), (1,)))
    q = qkv[:, :H * DH]
    k = qkv[:, H * DH:(H + KVH) * DH]
    v = qkv[:, (H + KVH) * DH:]
    q = _chunk_rmsnorm(q, qnw_ref[...])
    k = _chunk_rmsnorm(k, knw_ref[...])
    posf = pos_ref[...].astype(_f32)
    jj = jax.lax.broadcasted_iota(jnp.int32, (posf.shape[0], 32), 1)
    f = posf * jnp.exp(jj.astype(_f32) * (-LOG_THETA / 32.0))
    cs = jnp.cos(f)
    sn = jnp.sin(f)
    q = _rope(q, cs, sn)
    k = _rope(k, cs, sn)
    q_ref[...] = q.astype(_bf16)
    k_ref[...] = k.astype(_bf16)
    v_ref[...] = v.astype(_bf16)


BTP = 512


def _pre(pos2d, hidden, wqkv_b, qnw, knw, ilnw):
    return pl.pallas_call(
        _pre_body,
        grid=(T // BTP,),
        in_specs=[
            pl.BlockSpec((BTP, 1), lambda i: (i, 0)),
            pl.BlockSpec((BTP, D), lambda i: (i, 0)),
            pl.BlockSpec(((H + 2 * KVH) * DH, D), lambda i: (0, 0)),
            pl.BlockSpec((1, H * DH), lambda i: (0, 0)),
            pl.BlockSpec((1, KVH * DH), lambda i: (0, 0)),
            pl.BlockSpec((1, D), lambda i: (0, 0)),
        ],
        out_specs=[
            pl.BlockSpec((BTP, H * DH), lambda i: (i, 0)),
            pl.BlockSpec((BTP, KVH * DH), lambda i: (i, 0)),
            pl.BlockSpec((BTP, KVH * DH), lambda i: (i, 0)),
        ],
        out_shape=[
            jax.ShapeDtypeStruct((T, H * DH), _bf16),
            jax.ShapeDtypeStruct((T, KVH * DH), _bf16),
            jax.ShapeDtypeStruct((T, KVH * DH), _bf16),
        ],
        compiler_params=pltpu.CompilerParams(
            dimension_semantics=("parallel",)),
        interpret=INTERPRET,
    )(pos2d, hidden, wqkv_b, qnw, knw, ilnw)


SCALE = DH ** -0.5
NHP = 4
NPAIRS = NT * (NT + 1) // 2


CS = 1024


def _attn_body(q_ref, k_ref, v_ref, o_ref):
    qi = pl.program_id(0)
    qs = [jnp.concatenate(
        [q_ref[:, DH * 2 * t:DH * (2 * t + 1)],
         q_ref[:, DH * (2 * t + 1):DH * (2 * t + 2)]], axis=0) * 0.125
        for t in range(KVH)]

    def chunk(base, carry, mask):
        out = []
        for t in range(KVH):
            l_old, acc_old = carry[t]
            kh = k_ref[pl.ds(base, CS), DH * t:DH * (t + 1)]
            vh = v_ref[pl.ds(base, CS), DH * t:DH * (t + 1)]
            sc = _dot(qs[t], kh, ((1,), (1,)))
            if mask is not None:
                sc = sc + mask
            pm = jnp.exp(sc)
            l_new = l_old + jnp.sum(pm, axis=1, keepdims=True)
            acc_new = acc_old + _dot(pm.astype(_bf16), vh, ((1,), (0,)))
            out.append((l_new, acc_new))
        return tuple(out)

    init = tuple((jnp.zeros((2 * BT, 1), _f32), jnp.zeros((2 * BT, DH), _f32))
                 for _ in range(KVH))
    nfull = qi // (CS // BT)
    carry = jax.lax.fori_loop(
        0, nfull, lambda s, c: chunk(pl.multiple_of(s * CS, CS), c, None),
        init)
    rows = (jax.lax.broadcasted_iota(jnp.int32, (2 * BT, CS), 0) % BT
            + qi * BT)
    cols = jax.lax.broadcasted_iota(jnp.int32, (2 * BT, CS), 1) + nfull * CS
    pen = jnp.where(rows >= cols, 0.0, -1e9).astype(_f32)
    carry = chunk(pl.multiple_of(nfull * CS, CS), carry, pen)
    for t in range(KVH):
        l_f, acc_f = carry[t]
        res = (acc_f / l_f).astype(_bf16)
        o_ref[:, DH * 2 * t:DH * (2 * t + 1)] = res[:BT]
        o_ref[:, DH * (2 * t + 1):DH * (2 * t + 2)] = res[BT:]


def _attn(q, k, v):
    return pl.pallas_call(
        _attn_body,
        grid=(NT,),
        in_specs=[
            pl.BlockSpec((BT, H * DH), lambda i: (i, 0)),
            pl.BlockSpec((T, KVH * DH), lambda i: (0, 0)),
            pl.BlockSpec((T, KVH * DH), lambda i: (0, 0)),
        ],
        out_specs=pl.BlockSpec((BT, H * DH), lambda i: (i, 0)),
        out_shape=jax.ShapeDtypeStruct((T, H * DH), _bf16),
        compiler_params=pltpu.CompilerParams(
            dimension_semantics=("arbitrary",)),
        interpret=INTERPRET,
    )(q, k, v)


def _mid_body(hid_ref, attn_ref, wo_ref, plnw_ref, wr_ref,
              x_ref, m_ref, lg_ref):
    x = hid_ref[...] + _dot(attn_ref[...], wo_ref[...], ((1,), (1,)))
    x_ref[...] = x
    m = x * jax.lax.rsqrt(jnp.mean(x * x, axis=1, keepdims=True) + EPS)
    m = m * plnw_ref[...]
    m_ref[...] = m
    lg_ref[...] = _dot(m, wr_ref[...], ((1,), (1,)))


def _mid(hidden, attn, wo_b, plnw, wr_pad):
    return pl.pallas_call(
        _mid_body,
        grid=(NT,),
        in_specs=[
            pl.BlockSpec((BT, D), lambda i: (i, 0)),
            pl.BlockSpec((BT, H * DH), lambda i: (i, 0)),
            pl.BlockSpec((D, H * DH), lambda i: (0, 0)),
            pl.BlockSpec((1, D), lambda i: (0, 0)),
            pl.BlockSpec((128, D), lambda i: (0, 0)),
        ],
        out_specs=[
            pl.BlockSpec((BT, D), lambda i: (i, 0)),
            pl.BlockSpec((BT, D), lambda i: (i, 0)),
            pl.BlockSpec((BT, 128), lambda i: (i, 0)),
        ],
        out_shape=[
            jax.ShapeDtypeStruct((T, D), _f32),
            jax.ShapeDtypeStruct((T, D), _f32),
            jax.ShapeDtypeStruct((T, 128), _f32),
        ],
        compiler_params=pltpu.CompilerParams(
            dimension_semantics=("parallel",)),
        interpret=INTERPRET,
    )(hidden, attn, wo_b, plnw, wr_pad)


def _cumsum_rows(x):
    n = x.shape[0]
    sh = 1
    while sh < n:
        z = jnp.zeros((sh, x.shape[1]), x.dtype)
        x = x + jnp.concatenate([z, x[:n - sh]], axis=0)
        sh *= 2
    return x


def _cumsum_lanes(x):
    n = x.shape[1]
    sh = 1
    while sh < n:
        z = jnp.zeros((x.shape[0], sh), x.dtype)
        x = x + jnp.concatenate([z, x[:, :n - sh]], axis=1)
        sh *= 2
    return x


def _route_body(lg_ref, w0_ref, w1_ref, e0_ref, e1_ref, d0_ref, d1_ref,
                be_ref, bv_ref):
    lane = jax.lax.broadcasted_iota(jnp.int32, (T, 128), 1)
    valid = lane < E
    l = jnp.where(valid, lg_ref[...], -1e30)
    m0 = jnp.max(l, axis=1, keepdims=True)
    e0 = jnp.min(jnp.where(l == m0, lane, 9999), axis=1, keepdims=True)
    oh0 = lane == e0
    l1 = jnp.where(oh0, -1e30, l)
    m1 = jnp.max(l1, axis=1, keepdims=True)
    e1 = jnp.min(jnp.where(l1 == m1, lane, 9999), axis=1, keepdims=True)
    oh1 = lane == e1
    w0_ref[...] = jax.nn.sigmoid(m0 - m1)
    w1_ref[...] = jax.nn.sigmoid(m1 - m0)
    e0_ref[...] = e0
    e1_ref[...] = e1
    f0 = oh0.astype(_f32)
    f1 = oh1.astype(_f32)
    c0 = _cumsum_rows(f0)
    c1 = _cumsum_rows(f1)
    cnt0 = jnp.sum(f0, axis=0, keepdims=True)
    cnt1 = jnp.sum(f1, axis=0, keepdims=True)
    ncnt = cnt0 + cnt1
    ru = jnp.ceil(ncnt / BG) * BG
    off = _cumsum_lanes(ru) - ru
    rank0 = jnp.sum((c0 - f0) * f0, axis=1, keepdims=True)
    rank1 = jnp.sum((cnt0 + c1 - f1) * f1, axis=1, keepdims=True)
    base0 = jnp.sum(off * f0, axis=1, keepdims=True)
    base1 = jnp.sum(off * f1, axis=1, keepdims=True)
    d0_ref[...] = (base0 + rank0).astype(jnp.int32)
    d1_ref[...] = (base1 + rank1).astype(jnp.int32)
    jb = jax.lax.broadcasted_iota(jnp.int32, (32, 128), 0).astype(_f32)
    lane32 = jax.lax.broadcasted_iota(jnp.int32, (32, 128), 1)
    bstart = off / BG
    bend = bstart + ru / BG
    ine = ((jb >= bstart) & (jb < bend) & (lane32 < E)).astype(_f32)
    beval = jnp.sum(ine * lane32.astype(_f32), axis=1, keepdims=True)
    has = jnp.sum(ine, axis=1, keepdims=True)
    lane1 = jax.lax.broadcasted_iota(jnp.int32, (1, 128), 1)
    last_e = jnp.max(jnp.where((ncnt > 0) & (lane1 < E),
                               lane1.astype(_f32), -1.0),
                     axis=1, keepdims=True)
    be_ref[...] = jnp.where(has > 0, beval, last_e).astype(jnp.int32)
    bv_ref[...] = (has > 0).astype(jnp.int32)


def _route(logits):
    return pl.pallas_call(
        _route_body,
        out_shape=[
            jax.ShapeDtypeStruct((T, 1), _f32),
            jax.ShapeDtypeStruct((T, 1), _f32),
            jax.ShapeDtypeStruct((T, 1), jnp.int32),
            jax.ShapeDtypeStruct((T, 1), jnp.int32),
            jax.ShapeDtypeStruct((T, 1), jnp.int32),
            jax.ShapeDtypeStruct((T, 1), jnp.int32),
            jax.ShapeDtypeStruct((32, 1), jnp.int32),
            jax.ShapeDtypeStruct((32, 1), jnp.int32),
        ],
        interpret=INTERPRET,
    )(logits)


def _moe_dense_body(m_ref, wgu_ref, wd_ref, e0_ref, e1_ref, w0_ref, w1_ref,
                    out_ref):
    e = pl.program_id(1)
    mb = m_ref[...].astype(_bf16)
    gu = _dot(mb, wgu_ref[...][0], ((1,), (0,)))
    hh = jax.nn.silu(gu[:, :FF]) * gu[:, FF:]
    y = _dot(hh.astype(_bf16), wd_ref[...][0], ((1,), (0,)))
    we = (jnp.where(e0_ref[...] == e, w0_ref[...], 0.0)
          + jnp.where(e1_ref[...] == e, w1_ref[...], 0.0))

    @pl.when(e == 0)
    def _():
        out_ref[...] = we * y

    @pl.when(e > 0)
    def _():
        out_ref[...] += we * y


def _moe_dense(m, wgu_b, wd_b, e0, e1, w0, w1):
    return pl.pallas_call(
        _moe_dense_body,
        grid=(NT, E),
        in_specs=[
            pl.BlockSpec((BT, D), lambda i, e: (i, 0)),
            pl.BlockSpec((1, D, 2 * FF), lambda i, e: (e, 0, 0)),
            pl.BlockSpec((1, FF, D), lambda i, e: (e, 0, 0)),
            pl.BlockSpec((BT, 1), lambda i, e: (i, 0)),
            pl.BlockSpec((BT, 1), lambda i, e: (i, 0)),
            pl.BlockSpec((BT, 1), lambda i, e: (i, 0)),
            pl.BlockSpec((BT, 1), lambda i, e: (i, 0)),
        ],
        out_specs=pl.BlockSpec((BT, D), lambda i, e: (i, 0)),
        out_shape=jax.ShapeDtypeStruct((T, D), _f32),
        compiler_params=pltpu.CompilerParams(
            dimension_semantics=("parallel", "arbitrary")),
        interpret=INTERPRET,
    )(m, wgu_b, wd_b, e0, e1, w0, w1)


RPW = T // 32


def _sc_scatter(m, d0, d1):
    mesh = plsc.VectorSubcoreMesh(core_axis_name="c", subcore_axis_name="s")

    @functools.partial(
        pl.kernel, mesh=mesh,
        out_type=jax.ShapeDtypeStruct((NPAD, D), _f32),
        scratch_types=[
            pltpu.VMEM((RPW,), jnp.int32),
            pltpu.VMEM((RPW,), jnp.int32),
            pltpu.VMEM((RPW, D), _f32),
            pltpu.SemaphoreType.DMA,
        ],
    )
    def k(m_hbm, d0_hbm, d1_hbm, xs_hbm, idx0_v, idx1_v, rows_v, sem):
        wid = lax.axis_index("c") * 16 + lax.axis_index("s")
        base = wid * RPW
        pltpu.sync_copy(d0_hbm.at[pl.ds(base, RPW)], idx0_v)
        pltpu.sync_copy(d1_hbm.at[pl.ds(base, RPW)], idx1_v)
        pltpu.sync_copy(m_hbm.at[pl.ds(base, RPW)], rows_v)
        pltpu.async_copy(rows_v, xs_hbm.at[idx0_v], sem).wait()
        pltpu.async_copy(rows_v, xs_hbm.at[idx1_v], sem).wait()

    return k(m, d0, d1)


def _sc_gather(ys, d0, d1):
    mesh = plsc.VectorSubcoreMesh(core_axis_name="c", subcore_axis_name="s")

    @functools.partial(
        pl.kernel, mesh=mesh,
        out_type=[
            jax.ShapeDtypeStruct((T, D), _f32),
            jax.ShapeDtypeStruct((T, D), _f32),
        ],
        scratch_types=[
            pltpu.VMEM((RPW,), jnp.int32),
            pltpu.VMEM((RPW, D), _f32),
            pltpu.SemaphoreType.DMA,
        ],
    )
    def k(ys_hbm, d0_hbm, d1_hbm, y0_hbm, y1_hbm, idx_v, rows_v, sem):
        wid = lax.axis_index("c") * 16 + lax.axis_index("s")
        base = wid * RPW
        pltpu.sync_copy(d0_hbm.at[pl.ds(base, RPW)], idx_v)
        pltpu.async_copy(ys_hbm.at[idx_v], rows_v, sem).wait()
        pltpu.sync_copy(rows_v, y0_hbm.at[pl.ds(base, RPW)])
        pltpu.sync_copy(d1_hbm.at[pl.ds(base, RPW)], idx_v)
        pltpu.async_copy(ys_hbm.at[idx_v], rows_v, sem).wait()
        pltpu.sync_copy(rows_v, y1_hbm.at[pl.ds(base, RPW)])

    return k(ys, d0, d1)


def _group_body(be_ref, bv_ref, xs_ref, wgu_ref, wd_ref, ys_ref):
    i = pl.program_id(0)

    @pl.when(bv_ref[i] > 0)
    def _():
        mb = xs_ref[...].astype(_bf16)
        gu = _dot(mb, wgu_ref[...][0], ((1,), (0,)))
        hh = jax.nn.silu(gu[:, :FF]) * gu[:, FF:]
        ys_ref[...] = _dot(hh.astype(_bf16), wd_ref[...][0], ((1,), (0,)))


def _group(xs, wgu_b, wd_b, be, bv):
    grid_spec = pltpu.PrefetchScalarGridSpec(
        num_scalar_prefetch=2,
        grid=(MAXB,),
        in_specs=[
            pl.BlockSpec((BG, D),
                         lambda i, be, bv: (jnp.where(bv[i] > 0, i, 0), 0)),
            pl.BlockSpec((1, D, 2 * FF), lambda i, be, bv: (be[i], 0, 0)),
            pl.BlockSpec((1, FF, D), lambda i, be, bv: (be[i], 0, 0)),
        ],
        out_specs=pl.BlockSpec((BG, D), lambda i, be, bv: (i, 0)),
    )
    return pl.pallas_call(
        _group_body,
        grid_spec=grid_spec,
        out_shape=jax.ShapeDtypeStruct((NPAD, D), _f32),
        compiler_params=pltpu.CompilerParams(
            dimension_semantics=("arbitrary",)),
        interpret=INTERPRET,
    )(be, bv, xs, wgu_b, wd_b)


def _comb_body(x_ref, m_ref, y0_ref, y1_ref, w0_ref, w1_ref,
               wsgu_ref, wsd_ref, wsg_ref, out_ref):
    m = m_ref[...]
    sgu = _dot(m.astype(_bf16), wsgu_ref[...], ((1,), (0,)))
    sh = jax.nn.silu(sgu[:, :SFF]) * sgu[:, SFF:]
    sy = _dot(sh.astype(_bf16), wsd_ref[...], ((1,), (0,)))
    sg = jax.nn.sigmoid(jnp.sum(m * wsg_ref[...], axis=1, keepdims=True))
    fused = w0_ref[...] * y0_ref[...] + w1_ref[...] * y1_ref[...]
    out_ref[...] = x_ref[...] + fused + sg * sy


def _comb(x, m, y0, y1, w0, w1, wsgu_b, wsd_b, wsg):
    return pl.pallas_call(
        _comb_body,
        grid=(NT,),
        in_specs=[
            pl.BlockSpec((BT, D), lambda i: (i, 0)),
            pl.BlockSpec((BT, D), lambda i: (i, 0)),
            pl.BlockSpec((BT, D), lambda i: (i, 0)),
            pl.BlockSpec((BT, D), lambda i: (i, 0)),
            pl.BlockSpec((BT, 1), lambda i: (i, 0)),
            pl.BlockSpec((BT, 1), lambda i: (i, 0)),
            pl.BlockSpec((D, 2 * SFF), lambda i: (0, 0)),
            pl.BlockSpec((SFF, D), lambda i: (0, 0)),
            pl.BlockSpec((1, D), lambda i: (0, 0)),
        ],
        out_specs=pl.BlockSpec((BT, D), lambda i: (i, 0)),
        out_shape=jax.ShapeDtypeStruct((T, D), _f32),
        compiler_params=pltpu.CompilerParams(
            dimension_semantics=("parallel",)),
        interpret=INTERPRET,
    )(x, m, y0, y1, w0, w1, wsgu_b, wsd_b, wsg)


def kernel(positions, hidden_states, w_qkv, w_o, q_norm_w, k_norm_w,
           input_ln_w, post_ln_w, w_router, w_gate_up, w_down,
           w_shared_gu, w_shared_down, w_shared_gate):
    pos2d = positions.reshape(T, 1).astype(jnp.int32)
    qnw = jnp.tile(q_norm_w, H).reshape(1, H * DH)
    knw = jnp.tile(k_norm_w, KVH).reshape(1, KVH * DH)
    ilnw = input_ln_w.reshape(1, D)
    plnw = post_ln_w.reshape(1, D)
    wr_pad = jnp.pad(w_router, ((0, 128 - E), (0, 0)))
    wqkv_b = w_qkv.astype(_bf16)
    wo_b = w_o.astype(_bf16)
    wgu_b = w_gate_up.astype(_bf16)
    wd_b = w_down.astype(_bf16)
    wsgu_b = w_shared_gu.astype(_bf16)
    wsd_b = w_shared_down.astype(_bf16)
    wsg = w_shared_gate.reshape(1, D)

    q, k, v = _pre(pos2d, hidden_states, wqkv_b, qnw, knw, ilnw)
    attn = _attn(q, k, v)

    x, m, logits = _mid(hidden_states, attn, wo_b, plnw, wr_pad)
    w0, w1, e0, e1, d0, d1, be, bv = _route(logits)
    d0f = d0.reshape(T)
    d1f = d1.reshape(T)
    bef = be.reshape(32)[:MAXB]
    bvf = bv.reshape(32)[:MAXB]
    xs = _sc_scatter(m, d0f, d1f)
    ys = _group(xs, wgu_b, wd_b, bef, bvf)
    y0, y1 = _sc_gather(ys, d0f, d1f)
    out = _comb(x, m, y0, y1, w0, w1, wsgu_b, wsd_b, wsg)
    return out


# speedup vs baseline: 1.0501x; 1.0310x over previous
import functools

import jax
import jax.numpy as jnp
from jax import lax
from jax.experimental import pallas as pl
from jax.experimental.pallas import tpu as pltpu
from jax.experimental.pallas import tpu_sc as plsc

T = 2048
D = 1024
H = 16
KVH = 8
DH = 64
E = 8
TOPK = 2
FF = 512
SFF = 512
EPS = 1e-6
THETA = 1000000.0
LOG_THETA = 13.815510557964274

BT = 256
NT = T // BT
BG = 512
MAXB = 15
NPAD = MAXB * BG

INTERPRET = False

_f32 = jnp.float32
_bf16 = jnp.bfloat16


def _dot(a, b, dims):
    return jax.lax.dot_general(a, b, (dims, ((), ())),
                               preferred_element_type=_f32)


def _chunk_rmsnorm(x, w_full):
    w = x.shape[1]
    c = w // DH
    rr = jax.lax.broadcasted_iota(jnp.int32, (w, c), 0)
    cc = jax.lax.broadcasted_iota(jnp.int32, (w, c), 1)
    m = (rr // DH == cc).astype(_f32)
    ssq = _dot(x * x, m, ((1,), (0,)))
    scale = jax.lax.rsqrt(ssq / DH + EPS)
    scale_full = _dot(scale, m, ((1,), (1,)))
    return x * scale_full * w_full


def _rope(x, cs, sn):
    w = x.shape[1]
    lane = jax.lax.broadcasted_iota(jnp.int32, x.shape, 1)
    first = (lane % DH) < 32
    cosf = jnp.tile(cs, (1, w // 32))
    sinf = jnp.tile(sn, (1, w // 32))
    xp = jnp.concatenate([x[:, w - 32:], x[:, :w - 32]], axis=1)
    xm = jnp.concatenate([x[:, 32:], x[:, :32]], axis=1)
    return jnp.where(first, x * cosf - xm * sinf, xp * sinf + x * cosf)


def _pre_body(pos_ref, hid_ref, wqkv_ref, qnw_ref, knw_ref, ilnw_ref,
              q_ref, k_ref, v_ref):
    x = hid_ref[...]
    h = x * jax.lax.rsqrt(jnp.mean(x * x, axis=1, keepdims=True) + EPS)
    h = h * ilnw_ref[...]
    qkv = _dot(h.astype(_bf16), wqkv_ref[...], ((1,), (1,)))
    q = qkv[:, :H * DH]
    k = qkv[:, H * DH:(H + KVH) * DH]
    v = qkv[:, (H + KVH) * DH:]
    q = _chunk_rmsnorm(q, qnw_ref[...])
    k = _chunk_rmsnorm(k, knw_ref[...])
    posf = pos_ref[...].astype(_f32)
    jj = jax.lax.broadcasted_iota(jnp.int32, (posf.shape[0], 32), 1)
    f = posf * jnp.exp(jj.astype(_f32) * (-LOG_THETA / 32.0))
    cs = jnp.cos(f)
    sn = jnp.sin(f)
    q = _rope(q, cs, sn)
    k = _rope(k, cs, sn)
    q_ref[...] = q.astype(_bf16)
    k_ref[...] = k.astype(_bf16)
    v_ref[...] = v.astype(_bf16)


BTP = 512


def _pre(pos2d, hidden, wqkv_b, qnw, knw, ilnw):
    return pl.pallas_call(
        _pre_body,
        grid=(T // BTP,),
        in_specs=[
            pl.BlockSpec((BTP, 1), lambda i: (i, 0)),
            pl.BlockSpec((BTP, D), lambda i: (i, 0)),
            pl.BlockSpec(((H + 2 * KVH) * DH, D), lambda i: (0, 0)),
            pl.BlockSpec((1, H * DH), lambda i: (0, 0)),
            pl.BlockSpec((1, KVH * DH), lambda i: (0, 0)),
            pl.BlockSpec((1, D), lambda i: (0, 0)),
        ],
        out_specs=[
            pl.BlockSpec((BTP, H * DH), lambda i: (i, 0)),
            pl.BlockSpec((BTP, KVH * DH), lambda i: (i, 0)),
            pl.BlockSpec((BTP, KVH * DH), lambda i: (i, 0)),
        ],
        out_shape=[
            jax.ShapeDtypeStruct((T, H * DH), _bf16),
            jax.ShapeDtypeStruct((T, KVH * DH), _bf16),
            jax.ShapeDtypeStruct((T, KVH * DH), _bf16),
        ],
        compiler_params=pltpu.CompilerParams(
            dimension_semantics=("parallel",)),
        interpret=INTERPRET,
    )(pos2d, hidden, wqkv_b, qnw, knw, ilnw)


SCALE = DH ** -0.5
NHP = 4
NPAIRS = NT * (NT + 1) // 2


CS = 1024


def _attn_body(q_ref, k_ref, v_ref, o_ref):
    qi = pl.program_id(0)
    qs = [q_ref[:, DH * j:DH * (j + 1)] * 0.125 for j in range(H)]

    def chunk(base, carry, mask):
        out = []
        for j in range(H):
            kj = j // 2
            l_old, acc_old = carry[j]
            kh = k_ref[pl.ds(base, CS), DH * kj:DH * (kj + 1)]
            vh = v_ref[pl.ds(base, CS), DH * kj:DH * (kj + 1)]
            sc = _dot(qs[j], kh, ((1,), (1,)))
            if mask is not None:
                sc = sc + mask
            pm = jnp.exp(sc)
            l_new = l_old + jnp.sum(pm, axis=1, keepdims=True)
            acc_new = acc_old + _dot(pm.astype(_bf16), vh, ((1,), (0,)))
            out.append((l_new, acc_new))
        return tuple(out)

    init = tuple((jnp.zeros((BT, 1), _f32), jnp.zeros((BT, DH), _f32))
                 for _ in range(H))
    nfull = qi // (CS // BT)
    carry = jax.lax.fori_loop(
        0, nfull, lambda s, c: chunk(pl.multiple_of(s * CS, CS), c, None),
        init)
    rows = jax.lax.broadcasted_iota(jnp.int32, (BT, CS), 0) + qi * BT
    cols = jax.lax.broadcasted_iota(jnp.int32, (BT, CS), 1) + nfull * CS
    pen = jnp.where(rows >= cols, 0.0, -1e9).astype(_f32)
    carry = chunk(pl.multiple_of(nfull * CS, CS), carry, pen)
    for j in range(H):
        l_f, acc_f = carry[j]
        o_ref[:, DH * j:DH * (j + 1)] = (acc_f / l_f).astype(_bf16)


def _attn(q, k, v):
    return pl.pallas_call(
        _attn_body,
        grid=(NT,),
        in_specs=[
            pl.BlockSpec((BT, H * DH), lambda i: (i, 0)),
            pl.BlockSpec((T, KVH * DH), lambda i: (0, 0)),
            pl.BlockSpec((T, KVH * DH), lambda i: (0, 0)),
        ],
        out_specs=pl.BlockSpec((BT, H * DH), lambda i: (i, 0)),
        out_shape=jax.ShapeDtypeStruct((T, H * DH), _bf16),
        compiler_params=pltpu.CompilerParams(
            dimension_semantics=("arbitrary",)),
        interpret=INTERPRET,
    )(q, k, v)


def _mid_body(hid_ref, attn_ref, wo_ref, plnw_ref, wr_ref,
              x_ref, m_ref, lg_ref):
    x = hid_ref[...] + _dot(attn_ref[...], wo_ref[...], ((1,), (1,)))
    x_ref[...] = x
    m = x * jax.lax.rsqrt(jnp.mean(x * x, axis=1, keepdims=True) + EPS)
    m = m * plnw_ref[...]
    m_ref[...] = m
    lg_ref[...] = _dot(m, wr_ref[...], ((1,), (1,)))


def _mid(hidden, attn, wo_b, plnw, wr_pad):
    return pl.pallas_call(
        _mid_body,
        grid=(NT,),
        in_specs=[
            pl.BlockSpec((BT, D), lambda i: (i, 0)),
            pl.BlockSpec((BT, H * DH), lambda i: (i, 0)),
            pl.BlockSpec((D, H * DH), lambda i: (0, 0)),
            pl.BlockSpec((1, D), lambda i: (0, 0)),
            pl.BlockSpec((128, D), lambda i: (0, 0)),
        ],
        out_specs=[
            pl.BlockSpec((BT, D), lambda i: (i, 0)),
            pl.BlockSpec((BT, D), lambda i: (i, 0)),
            pl.BlockSpec((BT, 128), lambda i: (i, 0)),
        ],
        out_shape=[
            jax.ShapeDtypeStruct((T, D), _f32),
            jax.ShapeDtypeStruct((T, D), _f32),
            jax.ShapeDtypeStruct((T, 128), _f32),
        ],
        compiler_params=pltpu.CompilerParams(
            dimension_semantics=("parallel",)),
        interpret=INTERPRET,
    )(hidden, attn, wo_b, plnw, wr_pad)


def _cumsum_rows(x):
    n = x.shape[0]
    sh = 1
    while sh < n:
        z = jnp.zeros((sh, x.shape[1]), x.dtype)
        x = x + jnp.concatenate([z, x[:n - sh]], axis=0)
        sh *= 2
    return x


def _cumsum_lanes(x):
    n = x.shape[1]
    sh = 1
    while sh < n:
        z = jnp.zeros((x.shape[0], sh), x.dtype)
        x = x + jnp.concatenate([z, x[:, :n - sh]], axis=1)
        sh *= 2
    return x


def _route_body(lg_ref, w0_ref, w1_ref, e0_ref, e1_ref, d0_ref, d1_ref,
                be_ref, bv_ref):
    lane = jax.lax.broadcasted_iota(jnp.int32, (T, 128), 1)
    valid = lane < E
    l = jnp.where(valid, lg_ref[...], -1e30)
    m0 = jnp.max(l, axis=1, keepdims=True)
    e0 = jnp.min(jnp.where(l == m0, lane, 9999), axis=1, keepdims=True)
    oh0 = lane == e0
    l1 = jnp.where(oh0, -1e30, l)
    m1 = jnp.max(l1, axis=1, keepdims=True)
    e1 = jnp.min(jnp.where(l1 == m1, lane, 9999), axis=1, keepdims=True)
    oh1 = lane == e1
    w0_ref[...] = jax.nn.sigmoid(m0 - m1)
    w1_ref[...] = jax.nn.sigmoid(m1 - m0)
    e0_ref[...] = e0
    e1_ref[...] = e1
    f0 = oh0.astype(_f32)
    f1 = oh1.astype(_f32)
    c0 = _cumsum_rows(f0)
    c1 = _cumsum_rows(f1)
    cnt0 = jnp.sum(f0, axis=0, keepdims=True)
    cnt1 = jnp.sum(f1, axis=0, keepdims=True)
    ncnt = cnt0 + cnt1
    ru = jnp.ceil(ncnt / BG) * BG
    off = _cumsum_lanes(ru) - ru
    rank0 = jnp.sum((c0 - f0) * f0, axis=1, keepdims=True)
    rank1 = jnp.sum((cnt0 + c1 - f1) * f1, axis=1, keepdims=True)
    base0 = jnp.sum(off * f0, axis=1, keepdims=True)
    base1 = jnp.sum(off * f1, axis=1, keepdims=True)
    d0_ref[...] = (base0 + rank0).astype(jnp.int32)
    d1_ref[...] = (base1 + rank1).astype(jnp.int32)
    jb = jax.lax.broadcasted_iota(jnp.int32, (32, 128), 0).astype(_f32)
    lane32 = jax.lax.broadcasted_iota(jnp.int32, (32, 128), 1)
    bstart = off / BG
    bend = bstart + ru / BG
    ine = ((jb >= bstart) & (jb < bend) & (lane32 < E)).astype(_f32)
    beval = jnp.sum(ine * lane32.astype(_f32), axis=1, keepdims=True)
    has = jnp.sum(ine, axis=1, keepdims=True)
    lane1 = jax.lax.broadcasted_iota(jnp.int32, (1, 128), 1)
    last_e = jnp.max(jnp.where((ncnt > 0) & (lane1 < E),
                               lane1.astype(_f32), -1.0),
                     axis=1, keepdims=True)
    be_ref[...] = jnp.where(has > 0, beval, last_e).astype(jnp.int32)
    bv_ref[...] = (has > 0).astype(jnp.int32)


def _route(logits):
    return pl.pallas_call(
        _route_body,
        out_shape=[
            jax.ShapeDtypeStruct((T, 1), _f32),
            jax.ShapeDtypeStruct((T, 1), _f32),
            jax.ShapeDtypeStruct((T, 1), jnp.int32),
            jax.ShapeDtypeStruct((T, 1), jnp.int32),
            jax.ShapeDtypeStruct((T, 1), jnp.int32),
            jax.ShapeDtypeStruct((T, 1), jnp.int32),
            jax.ShapeDtypeStruct((32, 1), jnp.int32),
            jax.ShapeDtypeStruct((32, 1), jnp.int32),
        ],
        interpret=INTERPRET,
    )(logits)


def _moe_dense_body(m_ref, wgu_ref, wd_ref, e0_ref, e1_ref, w0_ref, w1_ref,
                    out_ref):
    e = pl.program_id(1)
    mb = m_ref[...].astype(_bf16)
    gu = _dot(mb, wgu_ref[...][0], ((1,), (0,)))
    hh = jax.nn.silu(gu[:, :FF]) * gu[:, FF:]
    y = _dot(hh.astype(_bf16), wd_ref[...][0], ((1,), (0,)))
    we = (jnp.where(e0_ref[...] == e, w0_ref[...], 0.0)
          + jnp.where(e1_ref[...] == e, w1_ref[...], 0.0))

    @pl.when(e == 0)
    def _():
        out_ref[...] = we * y

    @pl.when(e > 0)
    def _():
        out_ref[...] += we * y


def _moe_dense(m, wgu_b, wd_b, e0, e1, w0, w1):
    return pl.pallas_call(
        _moe_dense_body,
        grid=(NT, E),
        in_specs=[
            pl.BlockSpec((BT, D), lambda i, e: (i, 0)),
            pl.BlockSpec((1, D, 2 * FF), lambda i, e: (e, 0, 0)),
            pl.BlockSpec((1, FF, D), lambda i, e: (e, 0, 0)),
            pl.BlockSpec((BT, 1), lambda i, e: (i, 0)),
            pl.BlockSpec((BT, 1), lambda i, e: (i, 0)),
            pl.BlockSpec((BT, 1), lambda i, e: (i, 0)),
            pl.BlockSpec((BT, 1), lambda i, e: (i, 0)),
        ],
        out_specs=pl.BlockSpec((BT, D), lambda i, e: (i, 0)),
        out_shape=jax.ShapeDtypeStruct((T, D), _f32),
        compiler_params=pltpu.CompilerParams(
            dimension_semantics=("parallel", "arbitrary")),
        interpret=INTERPRET,
    )(m, wgu_b, wd_b, e0, e1, w0, w1)


RPW = T // 32


def _sc_scatter(m, d0, d1):
    mesh = plsc.VectorSubcoreMesh(core_axis_name="c", subcore_axis_name="s")

    @functools.partial(
        pl.kernel, mesh=mesh,
        out_type=jax.ShapeDtypeStruct((NPAD, D), _f32),
        scratch_types=[
            pltpu.VMEM((RPW,), jnp.int32),
            pltpu.VMEM((RPW,), jnp.int32),
            pltpu.VMEM((RPW, D), _f32),
            pltpu.SemaphoreType.DMA,
        ],
    )
    def k(m_hbm, d0_hbm, d1_hbm, xs_hbm, idx0_v, idx1_v, rows_v, sem):
        wid = lax.axis_index("c") * 16 + lax.axis_index("s")
        base = wid * RPW
        pltpu.sync_copy(d0_hbm.at[pl.ds(base, RPW)], idx0_v)
        pltpu.sync_copy(d1_hbm.at[pl.ds(base, RPW)], idx1_v)
        pltpu.sync_copy(m_hbm.at[pl.ds(base, RPW)], rows_v)
        pltpu.async_copy(rows_v, xs_hbm.at[idx0_v], sem).wait()
        pltpu.async_copy(rows_v, xs_hbm.at[idx1_v], sem).wait()

    return k(m, d0, d1)


def _sc_gather(ys, d0, d1):
    mesh = plsc.VectorSubcoreMesh(core_axis_name="c", subcore_axis_name="s")

    @functools.partial(
        pl.kernel, mesh=mesh,
        out_type=[
            jax.ShapeDtypeStruct((T, D), _f32),
            jax.ShapeDtypeStruct((T, D), _f32),
        ],
        scratch_types=[
            pltpu.VMEM((RPW,), jnp.int32),
            pltpu.VMEM((RPW, D), _f32),
            pltpu.SemaphoreType.DMA,
        ],
    )
    def k(ys_hbm, d0_hbm, d1_hbm, y0_hbm, y1_hbm, idx_v, rows_v, sem):
        wid = lax.axis_index("c") * 16 + lax.axis_index("s")
        base = wid * RPW
        pltpu.sync_copy(d0_hbm.at[pl.ds(base, RPW)], idx_v)
        pltpu.async_copy(ys_hbm.at[idx_v], rows_v, sem).wait()
        pltpu.sync_copy(rows_v, y0_hbm.at[pl.ds(base, RPW)])
        pltpu.sync_copy(d1_hbm.at[pl.ds(base, RPW)], idx_v)
        pltpu.async_copy(ys_hbm.at[idx_v], rows_v, sem).wait()
        pltpu.sync_copy(rows_v, y1_hbm.at[pl.ds(base, RPW)])

    return k(ys, d0, d1)


def _group_body(be_ref, bv_ref, xs_ref, wgu_ref, wd_ref, ys_ref):
    i = pl.program_id(0)

    @pl.when(bv_ref[i] > 0)
    def _():
        mb = xs_ref[...].astype(_bf16)
        gu = _dot(mb, wgu_ref[...][0], ((1,), (0,)))
        hh = jax.nn.silu(gu[:, :FF]) * gu[:, FF:]
        ys_ref[...] = _dot(hh.astype(_bf16), wd_ref[...][0], ((1,), (0,)))


def _group(xs, wgu_b, wd_b, be, bv):
    grid_spec = pltpu.PrefetchScalarGridSpec(
        num_scalar_prefetch=2,
        grid=(MAXB,),
        in_specs=[
            pl.BlockSpec((BG, D),
                         lambda i, be, bv: (jnp.where(bv[i] > 0, i, 0), 0)),
            pl.BlockSpec((1, D, 2 * FF), lambda i, be, bv: (be[i], 0, 0)),
            pl.BlockSpec((1, FF, D), lambda i, be, bv: (be[i], 0, 0)),
        ],
        out_specs=pl.BlockSpec((BG, D), lambda i, be, bv: (i, 0)),
    )
    return pl.pallas_call(
        _group_body,
        grid_spec=grid_spec,
        out_shape=jax.ShapeDtypeStruct((NPAD, D), _f32),
        compiler_params=pltpu.CompilerParams(
            dimension_semantics=("arbitrary",)),
        interpret=INTERPRET,
    )(be, bv, xs, wgu_b, wd_b)


def _comb_body(x_ref, m_ref, y0_ref, y1_ref, w0_ref, w1_ref,
               wsgu_ref, wsd_ref, wsg_ref, out_ref):
    m = m_ref[...]
    sgu = _dot(m.astype(_bf16), wsgu_ref[...], ((1,), (0,)))
    sh = jax.nn.silu(sgu[:, :SFF]) * sgu[:, SFF:]
    sy = _dot(sh.astype(_bf16), wsd_ref[...], ((1,), (0,)))
    sg = jax.nn.sigmoid(jnp.sum(m * wsg_ref[...], axis=1, keepdims=True))
    fused = w0_ref[...] * y0_ref[...] + w1_ref[...] * y1_ref[...]
    out_ref[...] = x_ref[...] + fused + sg * sy


def _comb(x, m, y0, y1, w0, w1, wsgu_b, wsd_b, wsg):
    return pl.pallas_call(
        _comb_body,
        grid=(NT,),
        in_specs=[
            pl.BlockSpec((BT, D), lambda i: (i, 0)),
            pl.BlockSpec((BT, D), lambda i: (i, 0)),
            pl.BlockSpec((BT, D), lambda i: (i, 0)),
            pl.BlockSpec((BT, D), lambda i: (i, 0)),
            pl.BlockSpec((BT, 1), lambda i: (i, 0)),
            pl.BlockSpec((BT, 1), lambda i: (i, 0)),
            pl.BlockSpec((D, 2 * SFF), lambda i: (0, 0)),
            pl.BlockSpec((SFF, D), lambda i: (0, 0)),
            pl.BlockSpec((1, D), lambda i: (0, 0)),
        ],
        out_specs=pl.BlockSpec((BT, D), lambda i: (i, 0)),
        out_shape=jax.ShapeDtypeStruct((T, D), _f32),
        compiler_params=pltpu.CompilerParams(
            dimension_semantics=("parallel",)),
        interpret=INTERPRET,
    )(x, m, y0, y1, w0, w1, wsgu_b, wsd_b, wsg)


def kernel(positions, hidden_states, w_qkv, w_o, q_norm_w, k_norm_w,
           input_ln_w, post_ln_w, w_router, w_gate_up, w_down,
           w_shared_gu, w_shared_down, w_shared_gate):
    pos2d = positions.reshape(T, 1).astype(jnp.int32)
    qnw = jnp.tile(q_norm_w, H).reshape(1, H * DH)
    knw = jnp.tile(k_norm_w, KVH).reshape(1, KVH * DH)
    ilnw = input_ln_w.reshape(1, D)
    plnw = post_ln_w.reshape(1, D)
    wr_pad = jnp.pad(w_router, ((0, 128 - E), (0, 0)))
    wqkv_b = w_qkv.astype(_bf16)
    wo_b = w_o.astype(_bf16)
    wgu_b = w_gate_up.astype(_bf16)
    wd_b = w_down.astype(_bf16)
    wsgu_b = w_shared_gu.astype(_bf16)
    wsd_b = w_shared_down.astype(_bf16)
    wsg = w_shared_gate.reshape(1, D)

    q, k, v = _pre(pos2d, hidden_states, wqkv_b, qnw, knw, ilnw)
    attn = _attn(q, k, v)

    x, m, logits = _mid(hidden_states, attn, wo_b, plnw, wr_pad)
    w0, w1, e0, e1, d0, d1, be, bv = _route(logits)
    d0f = d0.reshape(T)
    d1f = d1.reshape(T)
    bef = be.reshape(32)[:MAXB]
    bvf = bv.reshape(32)[:MAXB]
    xs = _sc_scatter(m, d0f, d1f)
    ys = _group(xs, wgu_b, wd_b, bef, bvf)
    y0, y1 = _sc_gather(ys, d0f, d1f)
    out = _comb(x, m, y0, y1, w0, w1, wsgu_b, wsd_b, wsg)
    return out


# speedup vs baseline: 1.0607x; 1.0101x over previous
import functools

import jax
import jax.numpy as jnp
from jax import lax
from jax.experimental import pallas as pl
from jax.experimental.pallas import tpu as pltpu
from jax.experimental.pallas import tpu_sc as plsc

T = 2048
D = 1024
H = 16
KVH = 8
DH = 64
E = 8
TOPK = 2
FF = 512
SFF = 512
EPS = 1e-6
THETA = 1000000.0
LOG_THETA = 13.815510557964274

BT = 256
NT = T // BT
BG = 512
MAXB = 15
NPAD = MAXB * BG

INTERPRET = False

_f32 = jnp.float32
_bf16 = jnp.bfloat16


def _dot(a, b, dims):
    return jax.lax.dot_general(a, b, (dims, ((), ())),
                               preferred_element_type=_f32)


def _chunk_rmsnorm(x, w_full):
    w = x.shape[1]
    c = w // DH
    rr = jax.lax.broadcasted_iota(jnp.int32, (w, c), 0)
    cc = jax.lax.broadcasted_iota(jnp.int32, (w, c), 1)
    m = (rr // DH == cc).astype(_f32)
    ssq = _dot(x * x, m, ((1,), (0,)))
    scale = jax.lax.rsqrt(ssq / DH + EPS)
    scale_full = _dot(scale, m, ((1,), (1,)))
    return x * scale_full * w_full


def _rope(x, cs, sn):
    w = x.shape[1]
    lane = jax.lax.broadcasted_iota(jnp.int32, x.shape, 1)
    first = (lane % DH) < 32
    cosf = jnp.tile(cs, (1, w // 32))
    sinf = jnp.tile(sn, (1, w // 32))
    xp = jnp.concatenate([x[:, w - 32:], x[:, :w - 32]], axis=1)
    xm = jnp.concatenate([x[:, 32:], x[:, :32]], axis=1)
    return jnp.where(first, x * cosf - xm * sinf, xp * sinf + x * cosf)


def _pre_body(pos_ref, hid_ref, wqkv_ref, qnw_ref, knw_ref, ilnw_ref,
              q_ref, k_ref, v_ref):
    x = hid_ref[...]
    h = x * jax.lax.rsqrt(jnp.mean(x * x, axis=1, keepdims=True) + EPS)
    h = h * ilnw_ref[...]
    qkv = _dot(h.astype(_bf16), wqkv_ref[...], ((1,), (1,)))
    q = qkv[:, :H * DH]
    k = qkv[:, H * DH:(H + KVH) * DH]
    v = qkv[:, (H + KVH) * DH:]
    q = _chunk_rmsnorm(q, qnw_ref[...])
    k = _chunk_rmsnorm(k, knw_ref[...])
    posf = pos_ref[...].astype(_f32)
    jj = jax.lax.broadcasted_iota(jnp.int32, (posf.shape[0], 32), 1)
    f = posf * jnp.exp(jj.astype(_f32) * (-LOG_THETA / 32.0))
    cs = jnp.cos(f)
    sn = jnp.sin(f)
    q = _rope(q, cs, sn)
    k = _rope(k, cs, sn)
    q_ref[...] = q.astype(_bf16)
    k_ref[...] = k.astype(_bf16)
    v_ref[...] = v.astype(_bf16)


BTP = 512


def _pre(pos2d, hidden, wqkv_b, qnw, knw, ilnw):
    return pl.pallas_call(
        _pre_body,
        grid=(T // BTP,),
        in_specs=[
            pl.BlockSpec((BTP, 1), lambda i: (i, 0)),
            pl.BlockSpec((BTP, D), lambda i: (i, 0)),
            pl.BlockSpec(((H + 2 * KVH) * DH, D), lambda i: (0, 0)),
            pl.BlockSpec((1, H * DH), lambda i: (0, 0)),
            pl.BlockSpec((1, KVH * DH), lambda i: (0, 0)),
            pl.BlockSpec((1, D), lambda i: (0, 0)),
        ],
        out_specs=[
            pl.BlockSpec((BTP, H * DH), lambda i: (i, 0)),
            pl.BlockSpec((BTP, KVH * DH), lambda i: (i, 0)),
            pl.BlockSpec((BTP, KVH * DH), lambda i: (i, 0)),
        ],
        out_shape=[
            jax.ShapeDtypeStruct((T, H * DH), _bf16),
            jax.ShapeDtypeStruct((T, KVH * DH), _bf16),
            jax.ShapeDtypeStruct((T, KVH * DH), _bf16),
        ],
        compiler_params=pltpu.CompilerParams(
            dimension_semantics=("parallel",)),
        interpret=INTERPRET,
    )(pos2d, hidden, wqkv_b, qnw, knw, ilnw)


SCALE = DH ** -0.5
NHP = 4
NPAIRS = NT * (NT + 1) // 2


CS = 1024


def _attn_body(q_ref, k_ref, v_ref, o_ref):
    qi = pl.program_id(0)
    qs = [q_ref[:, DH * j:DH * (j + 1)] * 0.125 for j in range(H)]

    def chunk(base, carry, mask):
        out = []
        for j in range(H):
            kj = j // 2
            l_old, acc_old = carry[j]
            kh = k_ref[pl.ds(base, CS), DH * kj:DH * (kj + 1)]
            vh = v_ref[pl.ds(base, CS), DH * kj:DH * (kj + 1)]
            sc = _dot(qs[j], kh, ((1,), (1,)))
            if mask is not None:
                sc = sc + mask
            pm = jnp.exp(sc)
            l_new = l_old + jnp.sum(pm, axis=1, keepdims=True)
            acc_new = acc_old + _dot(pm.astype(_bf16), vh, ((1,), (0,)))
            out.append((l_new, acc_new))
        return tuple(out)

    init = tuple((jnp.zeros((BT, 1), _f32), jnp.zeros((BT, DH), _f32))
                 for _ in range(H))
    nfull = qi // (CS // BT)
    carry = jax.lax.fori_loop(
        0, nfull, lambda s, c: chunk(pl.multiple_of(s * CS, CS), c, None),
        init)
    rows = jax.lax.broadcasted_iota(jnp.int32, (BT, CS), 0) + qi * BT
    cols = jax.lax.broadcasted_iota(jnp.int32, (BT, CS), 1) + nfull * CS
    pen = jnp.where(rows >= cols, 0.0, -1e9).astype(_f32)
    carry = chunk(pl.multiple_of(nfull * CS, CS), carry, pen)
    for j in range(H):
        l_f, acc_f = carry[j]
        o_ref[:, DH * j:DH * (j + 1)] = (acc_f / l_f).astype(_bf16)


def _attn(q, k, v):
    return pl.pallas_call(
        _attn_body,
        grid=(NT,),
        in_specs=[
            pl.BlockSpec((BT, H * DH), lambda i: (i, 0)),
            pl.BlockSpec((T, KVH * DH), lambda i: (0, 0)),
            pl.BlockSpec((T, KVH * DH), lambda i: (0, 0)),
        ],
        out_specs=pl.BlockSpec((BT, H * DH), lambda i: (i, 0)),
        out_shape=jax.ShapeDtypeStruct((T, H * DH), _bf16),
        compiler_params=pltpu.CompilerParams(
            dimension_semantics=("arbitrary",)),
        interpret=INTERPRET,
    )(q, k, v)


def _mid_body(hid_ref, attn_ref, wo_ref, plnw_ref, wr_ref,
              x_ref, m_ref, lg_ref):
    x = hid_ref[...] + _dot(attn_ref[...], wo_ref[...], ((1,), (1,)))
    x_ref[...] = x
    m = x * jax.lax.rsqrt(jnp.mean(x * x, axis=1, keepdims=True) + EPS)
    m = m * plnw_ref[...]
    m_ref[...] = m
    lg_ref[...] = _dot(m, wr_ref[...], ((1,), (1,)))


def _mid(hidden, attn, wo_b, plnw, wr_pad):
    return pl.pallas_call(
        _mid_body,
        grid=(T // BTP,),
        in_specs=[
            pl.BlockSpec((BTP, D), lambda i: (i, 0)),
            pl.BlockSpec((BTP, H * DH), lambda i: (i, 0)),
            pl.BlockSpec((D, H * DH), lambda i: (0, 0)),
            pl.BlockSpec((1, D), lambda i: (0, 0)),
            pl.BlockSpec((128, D), lambda i: (0, 0)),
        ],
        out_specs=[
            pl.BlockSpec((BTP, D), lambda i: (i, 0)),
            pl.BlockSpec((BTP, D), lambda i: (i, 0)),
            pl.BlockSpec((BTP, 128), lambda i: (i, 0)),
        ],
        out_shape=[
            jax.ShapeDtypeStruct((T, D), _f32),
            jax.ShapeDtypeStruct((T, D), _f32),
            jax.ShapeDtypeStruct((T, 128), _f32),
        ],
        compiler_params=pltpu.CompilerParams(
            dimension_semantics=("parallel",)),
        interpret=INTERPRET,
    )(hidden, attn, wo_b, plnw, wr_pad)


def _cumsum_rows(x):
    n = x.shape[0]
    sh = 1
    while sh < n:
        z = jnp.zeros((sh, x.shape[1]), x.dtype)
        x = x + jnp.concatenate([z, x[:n - sh]], axis=0)
        sh *= 2
    return x


def _cumsum_lanes(x):
    n = x.shape[1]
    sh = 1
    while sh < n:
        z = jnp.zeros((x.shape[0], sh), x.dtype)
        x = x + jnp.concatenate([z, x[:, :n - sh]], axis=1)
        sh *= 2
    return x


def _route_body(lg_ref, w0_ref, w1_ref, e0_ref, e1_ref, d0_ref, d1_ref,
                be_ref, bv_ref):
    lane = jax.lax.broadcasted_iota(jnp.int32, (T, 128), 1)
    valid = lane < E
    l = jnp.where(valid, lg_ref[...], -1e30)
    m0 = jnp.max(l, axis=1, keepdims=True)
    e0 = jnp.min(jnp.where(l == m0, lane, 9999), axis=1, keepdims=True)
    oh0 = lane == e0
    l1 = jnp.where(oh0, -1e30, l)
    m1 = jnp.max(l1, axis=1, keepdims=True)
    e1 = jnp.min(jnp.where(l1 == m1, lane, 9999), axis=1, keepdims=True)
    oh1 = lane == e1
    w0_ref[...] = jax.nn.sigmoid(m0 - m1)
    w1_ref[...] = jax.nn.sigmoid(m1 - m0)
    e0_ref[...] = e0
    e1_ref[...] = e1
    f0 = oh0.astype(_f32)
    f1 = oh1.astype(_f32)
    c0 = _cumsum_rows(f0)
    c1 = _cumsum_rows(f1)
    cnt0 = jnp.sum(f0, axis=0, keepdims=True)
    cnt1 = jnp.sum(f1, axis=0, keepdims=True)
    ncnt = cnt0 + cnt1
    ru = jnp.ceil(ncnt / BG) * BG
    off = _cumsum_lanes(ru) - ru
    rank0 = jnp.sum((c0 - f0) * f0, axis=1, keepdims=True)
    rank1 = jnp.sum((cnt0 + c1 - f1) * f1, axis=1, keepdims=True)
    base0 = jnp.sum(off * f0, axis=1, keepdims=True)
    base1 = jnp.sum(off * f1, axis=1, keepdims=True)
    d0_ref[...] = (base0 + rank0).astype(jnp.int32)
    d1_ref[...] = (base1 + rank1).astype(jnp.int32)
    jb = jax.lax.broadcasted_iota(jnp.int32, (32, 128), 0).astype(_f32)
    lane32 = jax.lax.broadcasted_iota(jnp.int32, (32, 128), 1)
    bstart = off / BG
    bend = bstart + ru / BG
    ine = ((jb >= bstart) & (jb < bend) & (lane32 < E)).astype(_f32)
    beval = jnp.sum(ine * lane32.astype(_f32), axis=1, keepdims=True)
    has = jnp.sum(ine, axis=1, keepdims=True)
    lane1 = jax.lax.broadcasted_iota(jnp.int32, (1, 128), 1)
    last_e = jnp.max(jnp.where((ncnt > 0) & (lane1 < E),
                               lane1.astype(_f32), -1.0),
                     axis=1, keepdims=True)
    be_ref[...] = jnp.where(has > 0, beval, last_e).astype(jnp.int32)
    bv_ref[...] = (has > 0).astype(jnp.int32)


def _route(logits):
    return pl.pallas_call(
        _route_body,
        out_shape=[
            jax.ShapeDtypeStruct((T, 1), _f32),
            jax.ShapeDtypeStruct((T, 1), _f32),
            jax.ShapeDtypeStruct((T, 1), jnp.int32),
            jax.ShapeDtypeStruct((T, 1), jnp.int32),
            jax.ShapeDtypeStruct((T, 1), jnp.int32),
            jax.ShapeDtypeStruct((T, 1), jnp.int32),
            jax.ShapeDtypeStruct((32, 1), jnp.int32),
            jax.ShapeDtypeStruct((32, 1), jnp.int32),
        ],
        interpret=INTERPRET,
    )(logits)


def _moe_dense_body(m_ref, wgu_ref, wd_ref, e0_ref, e1_ref, w0_ref, w1_ref,
                    out_ref):
    e = pl.program_id(1)
    mb = m_ref[...].astype(_bf16)
    gu = _dot(mb, wgu_ref[...][0], ((1,), (0,)))
    hh = jax.nn.silu(gu[:, :FF]) * gu[:, FF:]
    y = _dot(hh.astype(_bf16), wd_ref[...][0], ((1,), (0,)))
    we = (jnp.where(e0_ref[...] == e, w0_ref[...], 0.0)
          + jnp.where(e1_ref[...] == e, w1_ref[...], 0.0))

    @pl.when(e == 0)
    def _():
        out_ref[...] = we * y

    @pl.when(e > 0)
    def _():
        out_ref[...] += we * y


def _moe_dense(m, wgu_b, wd_b, e0, e1, w0, w1):
    return pl.pallas_call(
        _moe_dense_body,
        grid=(NT, E),
        in_specs=[
            pl.BlockSpec((BT, D), lambda i, e: (i, 0)),
            pl.BlockSpec((1, D, 2 * FF), lambda i, e: (e, 0, 0)),
            pl.BlockSpec((1, FF, D), lambda i, e: (e, 0, 0)),
            pl.BlockSpec((BT, 1), lambda i, e: (i, 0)),
            pl.BlockSpec((BT, 1), lambda i, e: (i, 0)),
            pl.BlockSpec((BT, 1), lambda i, e: (i, 0)),
            pl.BlockSpec((BT, 1), lambda i, e: (i, 0)),
        ],
        out_specs=pl.BlockSpec((BT, D), lambda i, e: (i, 0)),
        out_shape=jax.ShapeDtypeStruct((T, D), _f32),
        compiler_params=pltpu.CompilerParams(
            dimension_semantics=("parallel", "arbitrary")),
        interpret=INTERPRET,
    )(m, wgu_b, wd_b, e0, e1, w0, w1)


RPW = T // 32


def _sc_scatter(m, d0, d1):
    mesh = plsc.VectorSubcoreMesh(core_axis_name="c", subcore_axis_name="s")

    @functools.partial(
        pl.kernel, mesh=mesh,
        out_type=jax.ShapeDtypeStruct((NPAD, D), _f32),
        scratch_types=[
            pltpu.VMEM((RPW,), jnp.int32),
            pltpu.VMEM((RPW,), jnp.int32),
            pltpu.VMEM((RPW, D), _f32),
            pltpu.SemaphoreType.DMA,
        ],
    )
    def k(m_hbm, d0_hbm, d1_hbm, xs_hbm, idx0_v, idx1_v, rows_v, sem):
        wid = lax.axis_index("c") * 16 + lax.axis_index("s")
        base = wid * RPW
        pltpu.sync_copy(d0_hbm.at[pl.ds(base, RPW)], idx0_v)
        pltpu.sync_copy(d1_hbm.at[pl.ds(base, RPW)], idx1_v)
        pltpu.sync_copy(m_hbm.at[pl.ds(base, RPW)], rows_v)
        pltpu.async_copy(rows_v, xs_hbm.at[idx0_v], sem).wait()
        pltpu.async_copy(rows_v, xs_hbm.at[idx1_v], sem).wait()

    return k(m, d0, d1)


def _sc_gather(ys, d0, d1):
    mesh = plsc.VectorSubcoreMesh(core_axis_name="c", subcore_axis_name="s")

    @functools.partial(
        pl.kernel, mesh=mesh,
        out_type=[
            jax.ShapeDtypeStruct((T, D), _f32),
            jax.ShapeDtypeStruct((T, D), _f32),
        ],
        scratch_types=[
            pltpu.VMEM((RPW,), jnp.int32),
            pltpu.VMEM((RPW, D), _f32),
            pltpu.SemaphoreType.DMA,
        ],
    )
    def k(ys_hbm, d0_hbm, d1_hbm, y0_hbm, y1_hbm, idx_v, rows_v, sem):
        wid = lax.axis_index("c") * 16 + lax.axis_index("s")
        base = wid * RPW
        pltpu.sync_copy(d0_hbm.at[pl.ds(base, RPW)], idx_v)
        pltpu.async_copy(ys_hbm.at[idx_v], rows_v, sem).wait()
        pltpu.sync_copy(rows_v, y0_hbm.at[pl.ds(base, RPW)])
        pltpu.sync_copy(d1_hbm.at[pl.ds(base, RPW)], idx_v)
        pltpu.async_copy(ys_hbm.at[idx_v], rows_v, sem).wait()
        pltpu.sync_copy(rows_v, y1_hbm.at[pl.ds(base, RPW)])

    return k(ys, d0, d1)


def _group_body(be_ref, bv_ref, xs_ref, wgu_ref, wd_ref, ys_ref):
    i = pl.program_id(0)

    @pl.when(bv_ref[i] > 0)
    def _():
        mb = xs_ref[...].astype(_bf16)
        gu = _dot(mb, wgu_ref[...][0], ((1,), (0,)))
        hh = jax.nn.silu(gu[:, :FF]) * gu[:, FF:]
        ys_ref[...] = _dot(hh.astype(_bf16), wd_ref[...][0], ((1,), (0,)))


def _group(xs, wgu_b, wd_b, be, bv):
    grid_spec = pltpu.PrefetchScalarGridSpec(
        num_scalar_prefetch=2,
        grid=(MAXB,),
        in_specs=[
            pl.BlockSpec((BG, D),
                         lambda i, be, bv: (jnp.where(bv[i] > 0, i, 0), 0)),
            pl.BlockSpec((1, D, 2 * FF), lambda i, be, bv: (be[i], 0, 0)),
            pl.BlockSpec((1, FF, D), lambda i, be, bv: (be[i], 0, 0)),
        ],
        out_specs=pl.BlockSpec((BG, D), lambda i, be, bv: (i, 0)),
    )
    return pl.pallas_call(
        _group_body,
        grid_spec=grid_spec,
        out_shape=jax.ShapeDtypeStruct((NPAD, D), _f32),
        compiler_params=pltpu.CompilerParams(
            dimension_semantics=("arbitrary",)),
        interpret=INTERPRET,
    )(be, bv, xs, wgu_b, wd_b)


def _comb_body(x_ref, m_ref, y0_ref, y1_ref, w0_ref, w1_ref,
               wsgu_ref, wsd_ref, wsg_ref, out_ref):
    m = m_ref[...]
    sgu = _dot(m.astype(_bf16), wsgu_ref[...], ((1,), (0,)))
    sh = jax.nn.silu(sgu[:, :SFF]) * sgu[:, SFF:]
    sy = _dot(sh.astype(_bf16), wsd_ref[...], ((1,), (0,)))
    sg = jax.nn.sigmoid(jnp.sum(m * wsg_ref[...], axis=1, keepdims=True))
    fused = w0_ref[...] * y0_ref[...] + w1_ref[...] * y1_ref[...]
    out_ref[...] = x_ref[...] + fused + sg * sy


def _comb(x, m, y0, y1, w0, w1, wsgu_b, wsd_b, wsg):
    return pl.pallas_call(
        _comb_body,
        grid=(T // BTP,),
        in_specs=[
            pl.BlockSpec((BTP, D), lambda i: (i, 0)),
            pl.BlockSpec((BTP, D), lambda i: (i, 0)),
            pl.BlockSpec((BTP, D), lambda i: (i, 0)),
            pl.BlockSpec((BTP, D), lambda i: (i, 0)),
            pl.BlockSpec((BTP, 1), lambda i: (i, 0)),
            pl.BlockSpec((BTP, 1), lambda i: (i, 0)),
            pl.BlockSpec((D, 2 * SFF), lambda i: (0, 0)),
            pl.BlockSpec((SFF, D), lambda i: (0, 0)),
            pl.BlockSpec((1, D), lambda i: (0, 0)),
        ],
        out_specs=pl.BlockSpec((BTP, D), lambda i: (i, 0)),
        out_shape=jax.ShapeDtypeStruct((T, D), _f32),
        compiler_params=pltpu.CompilerParams(
            dimension_semantics=("parallel",)),
        interpret=INTERPRET,
    )(x, m, y0, y1, w0, w1, wsgu_b, wsd_b, wsg)


def kernel(positions, hidden_states, w_qkv, w_o, q_norm_w, k_norm_w,
           input_ln_w, post_ln_w, w_router, w_gate_up, w_down,
           w_shared_gu, w_shared_down, w_shared_gate):
    pos2d = positions.reshape(T, 1).astype(jnp.int32)
    qnw = jnp.tile(q_norm_w, H).reshape(1, H * DH)
    knw = jnp.tile(k_norm_w, KVH).reshape(1, KVH * DH)
    ilnw = input_ln_w.reshape(1, D)
    plnw = post_ln_w.reshape(1, D)
    wr_pad = jnp.pad(w_router, ((0, 128 - E), (0, 0)))
    wqkv_b = w_qkv.astype(_bf16)
    wo_b = w_o.astype(_bf16)
    wgu_b = w_gate_up.astype(_bf16)
    wd_b = w_down.astype(_bf16)
    wsgu_b = w_shared_gu.astype(_bf16)
    wsd_b = w_shared_down.astype(_bf16)
    wsg = w_shared_gate.reshape(1, D)

    q, k, v = _pre(pos2d, hidden_states, wqkv_b, qnw, knw, ilnw)
    attn = _attn(q, k, v)

    x, m, logits = _mid(hidden_states, attn, wo_b, plnw, wr_pad)
    w0, w1, e0, e1, d0, d1, be, bv = _route(logits)
    d0f = d0.reshape(T)
    d1f = d1.reshape(T)
    bef = be.reshape(32)[:MAXB]
    bvf = bv.reshape(32)[:MAXB]
    xs = _sc_scatter(m, d0f, d1f)
    ys = _group(xs, wgu_b, wd_b, bef, bvf)
    y0, y1 = _sc_gather(ys, d0f, d1f)
    out = _comb(x, m, y0, y1, w0, w1, wsgu_b, wsd_b, wsg)
    return out


# speedup vs baseline: 1.0619x; 1.0011x over previous
import functools

import jax
import jax.numpy as jnp
from jax import lax
from jax.experimental import pallas as pl
from jax.experimental.pallas import tpu as pltpu
from jax.experimental.pallas import tpu_sc as plsc

T = 2048
D = 1024
H = 16
KVH = 8
DH = 64
E = 8
TOPK = 2
FF = 512
SFF = 512
EPS = 1e-6
THETA = 1000000.0
LOG_THETA = 13.815510557964274

BT = 256
NT = T // BT
BG = 512
MAXB = 15
NPAD = MAXB * BG

INTERPRET = False

_f32 = jnp.float32
_bf16 = jnp.bfloat16


def _dot(a, b, dims):
    return jax.lax.dot_general(a, b, (dims, ((), ())),
                               preferred_element_type=_f32)


def _chunk_rmsnorm(x, w_full):
    w = x.shape[1]
    c = w // DH
    rr = jax.lax.broadcasted_iota(jnp.int32, (w, c), 0)
    cc = jax.lax.broadcasted_iota(jnp.int32, (w, c), 1)
    m = (rr // DH == cc).astype(_f32)
    ssq = _dot(x * x, m, ((1,), (0,)))
    scale = jax.lax.rsqrt(ssq / DH + EPS)
    scale_full = _dot(scale, m, ((1,), (1,)))
    return x * scale_full * w_full


def _rope(x, cs, sn):
    w = x.shape[1]
    lane = jax.lax.broadcasted_iota(jnp.int32, x.shape, 1)
    first = (lane % DH) < 32
    cosf = jnp.tile(cs, (1, w // 32))
    sinf = jnp.tile(sn, (1, w // 32))
    xp = jnp.concatenate([x[:, w - 32:], x[:, :w - 32]], axis=1)
    xm = jnp.concatenate([x[:, 32:], x[:, :32]], axis=1)
    return jnp.where(first, x * cosf - xm * sinf, xp * sinf + x * cosf)


def _pre_body(pos_ref, hid_ref, wqkv_ref, qnw_ref, knw_ref, ilnw_ref,
              q_ref, k_ref, v_ref):
    x = hid_ref[...]
    h = x * jax.lax.rsqrt(jnp.mean(x * x, axis=1, keepdims=True) + EPS)
    h = h * ilnw_ref[...]
    qkv = _dot(h.astype(_bf16), wqkv_ref[...], ((1,), (1,)))
    q = qkv[:, :H * DH]
    k = qkv[:, H * DH:(H + KVH) * DH]
    v = qkv[:, (H + KVH) * DH:]
    q = _chunk_rmsnorm(q, qnw_ref[...])
    k = _chunk_rmsnorm(k, knw_ref[...])
    posf = pos_ref[...].astype(_f32)
    jj = jax.lax.broadcasted_iota(jnp.int32, (posf.shape[0], 32), 1)
    f = posf * jnp.exp(jj.astype(_f32) * (-LOG_THETA / 32.0))
    cs = jnp.cos(f)
    sn = jnp.sin(f)
    q = _rope(q, cs, sn)
    k = _rope(k, cs, sn)
    q_ref[...] = q.astype(_bf16)
    k_ref[...] = k.astype(_bf16)
    v_ref[...] = v.astype(_bf16)


BTP = 512


def _pre(pos2d, hidden, wqkv_b, qnw, knw, ilnw):
    return pl.pallas_call(
        _pre_body,
        grid=(T // BTP,),
        in_specs=[
            pl.BlockSpec((BTP, 1), lambda i: (i, 0)),
            pl.BlockSpec((BTP, D), lambda i: (i, 0)),
            pl.BlockSpec(((H + 2 * KVH) * DH, D), lambda i: (0, 0)),
            pl.BlockSpec((1, H * DH), lambda i: (0, 0)),
            pl.BlockSpec((1, KVH * DH), lambda i: (0, 0)),
            pl.BlockSpec((1, D), lambda i: (0, 0)),
        ],
        out_specs=[
            pl.BlockSpec((BTP, H * DH), lambda i: (i, 0)),
            pl.BlockSpec((BTP, KVH * DH), lambda i: (i, 0)),
            pl.BlockSpec((BTP, KVH * DH), lambda i: (i, 0)),
        ],
        out_shape=[
            jax.ShapeDtypeStruct((T, H * DH), _bf16),
            jax.ShapeDtypeStruct((T, KVH * DH), _bf16),
            jax.ShapeDtypeStruct((T, KVH * DH), _bf16),
        ],
        compiler_params=pltpu.CompilerParams(
            dimension_semantics=("parallel",)),
        interpret=INTERPRET,
    )(pos2d, hidden, wqkv_b, qnw, knw, ilnw)


CS = 1024


def _attn_body(q_ref, k_ref, v_ref, o_ref):
    qi = pl.program_id(0)
    qs = [q_ref[:, DH * j:DH * (j + 1)] * 0.125 for j in range(H)]

    def chunk(base, carry, mask):
        out = []
        for j in range(H):
            kj = j // 2
            l_old, acc_old = carry[j]
            kh = k_ref[pl.ds(base, CS), DH * kj:DH * (kj + 1)]
            vh = v_ref[pl.ds(base, CS), DH * kj:DH * (kj + 1)]
            sc = _dot(qs[j], kh, ((1,), (1,)))
            if mask is not None:
                sc = sc + mask
            pm = jnp.exp(sc)
            l_new = l_old + jnp.sum(pm, axis=1, keepdims=True)
            acc_new = acc_old + _dot(pm.astype(_bf16), vh, ((1,), (0,)))
            out.append((l_new, acc_new))
        return tuple(out)

    init = tuple((jnp.zeros((BT, 1), _f32), jnp.zeros((BT, DH), _f32))
                 for _ in range(H))
    nfull = qi // (CS // BT)
    carry = jax.lax.fori_loop(
        0, nfull, lambda s, c: chunk(pl.multiple_of(s * CS, CS), c, None),
        init)
    rows = jax.lax.broadcasted_iota(jnp.int32, (BT, CS), 0) + qi * BT
    cols = jax.lax.broadcasted_iota(jnp.int32, (BT, CS), 1) + nfull * CS
    pen = jnp.where(rows >= cols, 0.0, -1e9).astype(_f32)
    carry = chunk(pl.multiple_of(nfull * CS, CS), carry, pen)
    for j in range(H):
        l_f, acc_f = carry[j]
        o_ref[:, DH * j:DH * (j + 1)] = (acc_f / l_f).astype(_bf16)


def _attn(q, k, v):
    return pl.pallas_call(
        _attn_body,
        grid=(NT,),
        in_specs=[
            pl.BlockSpec((BT, H * DH), lambda i: (i, 0)),
            pl.BlockSpec((T, KVH * DH), lambda i: (0, 0)),
            pl.BlockSpec((T, KVH * DH), lambda i: (0, 0)),
        ],
        out_specs=pl.BlockSpec((BT, H * DH), lambda i: (i, 0)),
        out_shape=jax.ShapeDtypeStruct((T, H * DH), _bf16),
        compiler_params=pltpu.CompilerParams(
            dimension_semantics=("arbitrary",)),
        interpret=INTERPRET,
    )(q, k, v)


def _mid_body(hid_ref, attn_ref, wo_ref, plnw_ref, wr_ref,
              x_ref, m_ref, lg_ref):
    x = hid_ref[...] + _dot(attn_ref[...], wo_ref[...], ((1,), (1,)))
    x_ref[...] = x
    m = x * jax.lax.rsqrt(jnp.mean(x * x, axis=1, keepdims=True) + EPS)
    m = m * plnw_ref[...]
    m_ref[...] = m
    lg_ref[...] = _dot(m, wr_ref[...], ((1,), (1,)))


def _mid(hidden, attn, wo_b, plnw, wr_pad):
    return pl.pallas_call(
        _mid_body,
        grid=(T // BTP,),
        in_specs=[
            pl.BlockSpec((BTP, D), lambda i: (i, 0)),
            pl.BlockSpec((BTP, H * DH), lambda i: (i, 0)),
            pl.BlockSpec((D, H * DH), lambda i: (0, 0)),
            pl.BlockSpec((1, D), lambda i: (0, 0)),
            pl.BlockSpec((128, D), lambda i: (0, 0)),
        ],
        out_specs=[
            pl.BlockSpec((BTP, D), lambda i: (i, 0)),
            pl.BlockSpec((BTP, D), lambda i: (i, 0)),
            pl.BlockSpec((BTP, 128), lambda i: (i, 0)),
        ],
        out_shape=[
            jax.ShapeDtypeStruct((T, D), _f32),
            jax.ShapeDtypeStruct((T, D), _f32),
            jax.ShapeDtypeStruct((T, 128), _f32),
        ],
        compiler_params=pltpu.CompilerParams(
            dimension_semantics=("parallel",)),
        interpret=INTERPRET,
    )(hidden, attn, wo_b, plnw, wr_pad)


def _cumsum_rows(x):
    n = x.shape[0]
    sh = 1
    while sh < n:
        z = jnp.zeros((sh, x.shape[1]), x.dtype)
        x = x + jnp.concatenate([z, x[:n - sh]], axis=0)
        sh *= 2
    return x


def _cumsum_lanes(x):
    n = x.shape[1]
    sh = 1
    while sh < n:
        z = jnp.zeros((x.shape[0], sh), x.dtype)
        x = x + jnp.concatenate([z, x[:, :n - sh]], axis=1)
        sh *= 2
    return x


def _route_body(lg_ref, w0_ref, w1_ref, e0_ref, e1_ref, d0_ref, d1_ref,
                be_ref, bv_ref):
    lane = jax.lax.broadcasted_iota(jnp.int32, (T, 128), 1)
    valid = lane < E
    l = jnp.where(valid, lg_ref[...], -1e30)
    m0 = jnp.max(l, axis=1, keepdims=True)
    e0 = jnp.min(jnp.where(l == m0, lane, 9999), axis=1, keepdims=True)
    oh0 = lane == e0
    l1 = jnp.where(oh0, -1e30, l)
    m1 = jnp.max(l1, axis=1, keepdims=True)
    e1 = jnp.min(jnp.where(l1 == m1, lane, 9999), axis=1, keepdims=True)
    oh1 = lane == e1
    w0_ref[...] = jax.nn.sigmoid(m0 - m1)
    w1_ref[...] = jax.nn.sigmoid(m1 - m0)
    e0_ref[...] = e0
    e1_ref[...] = e1
    f0 = oh0.astype(_f32)
    f1 = oh1.astype(_f32)
    c0 = _cumsum_rows(f0)
    c1 = _cumsum_rows(f1)
    cnt0 = jnp.sum(f0, axis=0, keepdims=True)
    cnt1 = jnp.sum(f1, axis=0, keepdims=True)
    ncnt = cnt0 + cnt1
    ru = jnp.ceil(ncnt / BG) * BG
    off = _cumsum_lanes(ru) - ru
    rank0 = jnp.sum((c0 - f0) * f0, axis=1, keepdims=True)
    rank1 = jnp.sum((cnt0 + c1 - f1) * f1, axis=1, keepdims=True)
    base0 = jnp.sum(off * f0, axis=1, keepdims=True)
    base1 = jnp.sum(off * f1, axis=1, keepdims=True)
    d0_ref[...] = (base0 + rank0).astype(jnp.int32)
    d1_ref[...] = (base1 + rank1).astype(jnp.int32)
    jb = jax.lax.broadcasted_iota(jnp.int32, (32, 128), 0).astype(_f32)
    lane32 = jax.lax.broadcasted_iota(jnp.int32, (32, 128), 1)
    bstart = off / BG
    bend = bstart + ru / BG
    ine = ((jb >= bstart) & (jb < bend) & (lane32 < E)).astype(_f32)
    beval = jnp.sum(ine * lane32.astype(_f32), axis=1, keepdims=True)
    has = jnp.sum(ine, axis=1, keepdims=True)
    lane1 = jax.lax.broadcasted_iota(jnp.int32, (1, 128), 1)
    last_e = jnp.max(jnp.where((ncnt > 0) & (lane1 < E),
                               lane1.astype(_f32), -1.0),
                     axis=1, keepdims=True)
    be_ref[...] = jnp.where(has > 0, beval, last_e).astype(jnp.int32)
    bv_ref[...] = (has > 0).astype(jnp.int32)


def _route(logits):
    return pl.pallas_call(
        _route_body,
        out_shape=[
            jax.ShapeDtypeStruct((T, 1), _f32),
            jax.ShapeDtypeStruct((T, 1), _f32),
            jax.ShapeDtypeStruct((T, 1), jnp.int32),
            jax.ShapeDtypeStruct((T, 1), jnp.int32),
            jax.ShapeDtypeStruct((T, 1), jnp.int32),
            jax.ShapeDtypeStruct((T, 1), jnp.int32),
            jax.ShapeDtypeStruct((32, 1), jnp.int32),
            jax.ShapeDtypeStruct((32, 1), jnp.int32),
        ],
        interpret=INTERPRET,
    )(logits)


RPW = T // 32


def _sc_scatter(m, d0, d1):
    mesh = plsc.VectorSubcoreMesh(core_axis_name="c", subcore_axis_name="s")

    @functools.partial(
        pl.kernel, mesh=mesh,
        out_type=jax.ShapeDtypeStruct((NPAD, D), _f32),
        scratch_types=[
            pltpu.VMEM((RPW,), jnp.int32),
            pltpu.VMEM((RPW,), jnp.int32),
            pltpu.VMEM((RPW, D), _f32),
            pltpu.SemaphoreType.DMA,
        ],
    )
    def k(m_hbm, d0_hbm, d1_hbm, xs_hbm, idx0_v, idx1_v, rows_v, sem):
        wid = lax.axis_index("c") * 16 + lax.axis_index("s")
        base = wid * RPW
        pltpu.sync_copy(d0_hbm.at[pl.ds(base, RPW)], idx0_v)
        pltpu.sync_copy(d1_hbm.at[pl.ds(base, RPW)], idx1_v)
        pltpu.sync_copy(m_hbm.at[pl.ds(base, RPW)], rows_v)
        pltpu.async_copy(rows_v, xs_hbm.at[idx0_v], sem).wait()
        pltpu.async_copy(rows_v, xs_hbm.at[idx1_v], sem).wait()

    return k(m, d0, d1)


def _sc_gather(ys, d0, d1):
    mesh = plsc.VectorSubcoreMesh(core_axis_name="c", subcore_axis_name="s")

    @functools.partial(
        pl.kernel, mesh=mesh,
        out_type=[
            jax.ShapeDtypeStruct((T, D), _f32),
            jax.ShapeDtypeStruct((T, D), _f32),
        ],
        scratch_types=[
            pltpu.VMEM((RPW,), jnp.int32),
            pltpu.VMEM((RPW, D), _f32),
            pltpu.SemaphoreType.DMA,
        ],
    )
    def k(ys_hbm, d0_hbm, d1_hbm, y0_hbm, y1_hbm, idx_v, rows_v, sem):
        wid = lax.axis_index("c") * 16 + lax.axis_index("s")
        base = wid * RPW
        pltpu.sync_copy(d0_hbm.at[pl.ds(base, RPW)], idx_v)
        pltpu.async_copy(ys_hbm.at[idx_v], rows_v, sem).wait()
        pltpu.sync_copy(rows_v, y0_hbm.at[pl.ds(base, RPW)])
        pltpu.sync_copy(d1_hbm.at[pl.ds(base, RPW)], idx_v)
        pltpu.async_copy(ys_hbm.at[idx_v], rows_v, sem).wait()
        pltpu.sync_copy(rows_v, y1_hbm.at[pl.ds(base, RPW)])

    return k(ys, d0, d1)


def _group_body(be_ref, bv_ref, xs_ref, wgu_ref, wd_ref, ys_ref):
    i = pl.program_id(0)

    @pl.when(bv_ref[i] > 0)
    def _():
        mb = xs_ref[...].astype(_bf16)
        gu = _dot(mb, wgu_ref[...][0], ((1,), (0,)))
        hh = jax.nn.silu(gu[:, :FF]) * gu[:, FF:]
        ys_ref[...] = _dot(hh.astype(_bf16), wd_ref[...][0], ((1,), (0,)))


def _group(xs, wgu_b, wd_b, be, bv):
    grid_spec = pltpu.PrefetchScalarGridSpec(
        num_scalar_prefetch=2,
        grid=(MAXB,),
        in_specs=[
            pl.BlockSpec((BG, D),
                         lambda i, be, bv: (jnp.where(bv[i] > 0, i, 0), 0)),
            pl.BlockSpec((1, D, 2 * FF), lambda i, be, bv: (be[i], 0, 0)),
            pl.BlockSpec((1, FF, D), lambda i, be, bv: (be[i], 0, 0)),
        ],
        out_specs=pl.BlockSpec((BG, D), lambda i, be, bv: (i, 0)),
    )
    return pl.pallas_call(
        _group_body,
        grid_spec=grid_spec,
        out_shape=jax.ShapeDtypeStruct((NPAD, D), _f32),
        compiler_params=pltpu.CompilerParams(
            dimension_semantics=("arbitrary",)),
        interpret=INTERPRET,
    )(be, bv, xs, wgu_b, wd_b)


def _comb_body(x_ref, m_ref, y0_ref, y1_ref, w0_ref, w1_ref,
               wsgu_ref, wsd_ref, wsg_ref, out_ref):
    m = m_ref[...]
    sgu = _dot(m.astype(_bf16), wsgu_ref[...], ((1,), (0,)))
    sh = jax.nn.silu(sgu[:, :SFF]) * sgu[:, SFF:]
    sy = _dot(sh.astype(_bf16), wsd_ref[...], ((1,), (0,)))
    sg = jax.nn.sigmoid(jnp.sum(m * wsg_ref[...], axis=1, keepdims=True))
    fused = w0_ref[...] * y0_ref[...] + w1_ref[...] * y1_ref[...]
    out_ref[...] = x_ref[...] + fused + sg * sy


def _comb(x, m, y0, y1, w0, w1, wsgu_b, wsd_b, wsg):
    return pl.pallas_call(
        _comb_body,
        grid=(T // BTP,),
        in_specs=[
            pl.BlockSpec((BTP, D), lambda i: (i, 0)),
            pl.BlockSpec((BTP, D), lambda i: (i, 0)),
            pl.BlockSpec((BTP, D), lambda i: (i, 0)),
            pl.BlockSpec((BTP, D), lambda i: (i, 0)),
            pl.BlockSpec((BTP, 1), lambda i: (i, 0)),
            pl.BlockSpec((BTP, 1), lambda i: (i, 0)),
            pl.BlockSpec((D, 2 * SFF), lambda i: (0, 0)),
            pl.BlockSpec((SFF, D), lambda i: (0, 0)),
            pl.BlockSpec((1, D), lambda i: (0, 0)),
        ],
        out_specs=pl.BlockSpec((BTP, D), lambda i: (i, 0)),
        out_shape=jax.ShapeDtypeStruct((T, D), _f32),
        compiler_params=pltpu.CompilerParams(
            dimension_semantics=("parallel",)),
        interpret=INTERPRET,
    )(x, m, y0, y1, w0, w1, wsgu_b, wsd_b, wsg)


def kernel(positions, hidden_states, w_qkv, w_o, q_norm_w, k_norm_w,
           input_ln_w, post_ln_w, w_router, w_gate_up, w_down,
           w_shared_gu, w_shared_down, w_shared_gate):
    pos2d = positions.reshape(T, 1).astype(jnp.int32)
    qnw = jnp.tile(q_norm_w, H).reshape(1, H * DH)
    knw = jnp.tile(k_norm_w, KVH).reshape(1, KVH * DH)
    ilnw = input_ln_w.reshape(1, D)
    plnw = post_ln_w.reshape(1, D)
    wr_pad = jnp.pad(w_router, ((0, 128 - E), (0, 0)))
    wqkv_b = w_qkv.astype(_bf16)
    wo_b = w_o.astype(_bf16)
    wgu_b = w_gate_up.astype(_bf16)
    wd_b = w_down.astype(_bf16)
    wsgu_b = w_shared_gu.astype(_bf16)
    wsd_b = w_shared_down.astype(_bf16)
    wsg = w_shared_gate.reshape(1, D)

    q, k, v = _pre(pos2d, hidden_states, wqkv_b, qnw, knw, ilnw)
    attn = _attn(q, k, v)

    x, m, logits = _mid(hidden_states, attn, wo_b, plnw, wr_pad)
    w0, w1, e0, e1, d0, d1, be, bv = _route(logits)
    d0f = d0.reshape(T)
    d1f = d1.reshape(T)
    bef = be.reshape(32)[:MAXB]
    bvf = bv.reshape(32)[:MAXB]
    xs = _sc_scatter(m, d0f, d1f)
    ys = _group(xs, wgu_b, wd_b, bef, bvf)
    y0, y1 = _sc_gather(ys, d0f, d1f)
    out = _comb(x, m, y0, y1, w0, w1, wsgu_b, wsd_b, wsg)
    return out
